# baseline (speedup 1.0000x reference)
import jax
import jax.numpy as jnp
from jax import lax
from jax.experimental import pallas as pl
from jax.experimental.pallas import tpu as pltpu
from jax.experimental.pallas import tpu_sc as plsc

N = 10000
D = 128
E = 320000
NC, NS, L = 2, 16, 16
NW = NC * NS
NP = 10240
RPT = NP // NS
B = 128
ET = 331776
EPW_ES = ET // NW
EPW_FS = ET // NS
NCH_ES = EPW_ES // B
NCH_FS = EPW_FS // B

_f32 = jnp.float32
_i32 = jnp.int32

_SC_PARAMS = pltpu.CompilerParams(use_tc_tiling_on_sc=False)


def _mesh():
    return plsc.VectorSubcoreMesh(core_axis_name="c", subcore_axis_name="s",
                                  num_cores=NC, num_subcores=NS)


def _lane_take(x, idx):
    dn = lax.GatherDimensionNumbers(offset_dims=(), collapsed_slice_dims=(0,),
                                    start_index_map=(0,))
    return lax.gather(x, idx[:, None], dn, (1,),
                      mode=lax.GatherScatterMode.PROMISE_IN_BOUNDS)


def _bc16(v):
    return jnp.zeros((16,), _i32) + v


def _deg_body(dst_hbm, zeros8, ones8, deg_out, dst_v, row_v, acc_s, sem):
    c = lax.axis_index("c")
    s = lax.axis_index("s")
    w = c * NS + s
    for k in range(RPT // B):
        pltpu.sync_copy(zeros8, acc_s.at[pl.ds(s * RPT + k * B, B)])
    plsc.subcore_barrier()
    pltpu.sync_copy(ones8, row_v)

    def chunk(i, carry):
        base = w * EPW_ES + i * B
        pltpu.sync_copy(dst_hbm.at[pl.ds(base, B)], dst_v)
        pltpu.sync_copy(row_v, acc_s.at[dst_v], add=True)
        return carry

    lax.fori_loop(0, NCH_ES, chunk, 0)
    plsc.subcore_barrier()
    pltpu.sync_copy(acc_s.at[pl.ds(s * RPT, RPT)],
                    deg_out.at[pl.ds(c * NP + s * RPT, RPT)])


def _deg_call(dst_pad, zeros8, ones8):
    k = pl.kernel(
        _deg_body,
        out_type=jax.ShapeDtypeStruct((NC * NP, 8), _f32),
        mesh=_mesh(),
        scratch_types=[
            pltpu.VMEM((B,), _i32),
            pltpu.VMEM((B, 8), _f32),
            pltpu.VMEM_SHARED((NP, 8), _f32),
            pltpu.SemaphoreType.DMA,
        ],
        compiler_params=_SC_PARAMS,
    )
    return k(dst_pad, zeros8, ones8)


def _gcn_body(src_hbm, dst_hbm, gtab, zeros16, acc_out,
              src_t, dst_t, rows0, rows1, acc_s, sg0, sg1, ss0, ss1):
    c = lax.axis_index("c")
    s = lax.axis_index("s")
    w = c * NS + s
    rows = (rows0, rows1)
    sg = (sg0, sg1)
    ss = (ss0, ss1)
    for k in range(RPT // B):
        pltpu.sync_copy(zeros16, acc_s.at[pl.ds(s * RPT + k * B, B)])
    pltpu.sync_copy(src_hbm.at[pl.ds(w * EPW_ES, EPW_ES)], src_t)
    pltpu.sync_copy(dst_hbm.at[pl.ds(w * EPW_ES, EPW_ES)], dst_t)
    plsc.subcore_barrier()

    def g_desc(ch, b):
        return pltpu.make_async_copy(gtab.at[src_t.at[pl.ds(ch * B, B)]],
                                     rows[b], sg[b])

    def s_desc(ch, b):
        return pltpu.make_async_copy(
            rows[b], acc_s.at[dst_t.at[pl.ds(ch * B, B)]], ss[b])

    def step(ch, b, issue_next, first=False):
        if not first:
            s_desc(ch - 1, 1 - b).wait()
        if issue_next:
            g_desc(ch + 1, 1 - b).start()
        g_desc(ch, b).wait()
        pltpu.async_copy(rows[b], acc_s.at[dst_t.at[pl.ds(ch * B, B)]],
                         ss[b], add=True)

    g_desc(0, 0).start()

    def pair(g, carry):
        @pl.when(g == 0)
        def _():
            g_desc(1, 1).start()
            g_desc(0, 0).wait()
            pltpu.async_copy(rows[0], acc_s.at[dst_t.at[pl.ds(0, B)]],
                             ss[0], add=True)

        @pl.when(g > 0)
        def _():
            step(2 * g, 0, True)

        step(2 * g + 1, 1, True)
        return carry

    lax.fori_loop(0, NCH_ES // 2, pair, 0)
    step(NCH_ES - 1, 0, False)
    s_desc(NCH_ES - 1, 0).wait()
    plsc.subcore_barrier()
    pltpu.sync_copy(acc_s.at[pl.ds(s * RPT, RPT)],
                    acc_out.at[pl.ds(c * NP + s * RPT, RPT)])


def _gcn_call(src_pad, dst_pad, gtab, zeros16):
    k = pl.kernel(
        _gcn_body,
        out_type=jax.ShapeDtypeStruct((NC * NP, 16), _f32),
        mesh=_mesh(),
        scratch_types=[
            pltpu.VMEM((EPW_ES,), _i32),
            pltpu.VMEM((EPW_ES,), _i32),
            pltpu.VMEM((B, 16), _f32),
            pltpu.VMEM((B, 16), _f32),
            pltpu.VMEM_SHARED((NP, 16), _f32),
            pltpu.SemaphoreType.DMA,
            pltpu.SemaphoreType.DMA,
            pltpu.SemaphoreType.DMA,
            pltpu.SemaphoreType.DMA,
        ],
        compiler_params=_SC_PARAMS,
    )
    return k(src_pad, dst_pad, gtab, zeros16)


def _gat_body(gsrc_hbm, dst_hbm, htab, altabd, shift_hbm, zeros72, acc_out,
              gidx_t, dst_t, rows0, rows1, drow0, drow1, send0, send1,
              shift_v, acc_s, sg0, sg1, ss0, ss1):
    c = lax.axis_index("c")
    s = lax.axis_index("s")
    rows = (rows0, rows1)
    drow = (drow0, drow1)
    send = (send0, send1)
    sg = (sg0, sg1)
    ss = (ss0, ss1)
    for k in range(RPT // B):
        pltpu.sync_copy(zeros72, acc_s.at[pl.ds(s * RPT + k * B, B)])
    pltpu.sync_copy(gsrc_hbm.at[pl.ds(c * ET + s * EPW_FS, EPW_FS)], gidx_t)
    pltpu.sync_copy(dst_hbm.at[pl.ds(s * EPW_FS, EPW_FS)], dst_t)
    pltpu.sync_copy(shift_hbm, shift_v)
    plsc.subcore_barrier()
    shv = shift_v[...]

    def g_pair(ch, b):
        return (pltpu.make_async_copy(htab.at[gidx_t.at[pl.ds(ch * B, B)]],
                                      rows[b], sg[b]),
                pltpu.make_async_copy(altabd.at[dst_t.at[pl.ds(ch * B, B)]],
                                      drow[b], sg[b]))

    def g_issue(ch, b):
        for d in g_pair(ch, b):
            d.start()

    def g_wait(ch, b):
        for d in g_pair(ch, b):
            d.wait()

    def s_desc(ch, b):
        return pltpu.make_async_copy(
            send[b], acc_s.at[dst_t.at[pl.ds(ch * B, B)]], ss[b])

    def compute(b):
        for jj in range(B):
            sr = rows[b][jj, pl.ds(56, 16)]
            dr = drow[b][jj, :]
            e = sr + dr
            e = jnp.maximum(e, 0.2 * e)
            ex = jnp.exp(e - shv)
            send[b][jj, pl.ds(56, 16)] = ex
            for h in range(4):
                exb = _lane_take(ex, _bc16(8 + 4 * c + h))
                send[b][jj, pl.ds(16 * h, 16)] = (
                    rows[b][jj, pl.ds(16 * h, 16)] * exb)

    g_issue(0, 0)

    def pair(g, carry):
        g_issue(2 * g + 1, 1)
        g_wait(2 * g, 0)

        @pl.when(g >= 1)
        def _():
            s_desc(2 * g - 2, 0).wait()

        compute(0)
        pltpu.async_copy(send[0], acc_s.at[dst_t.at[pl.ds((2 * g) * B, B)]],
                         ss[0], add=True)
        @pl.when(g < NCH_FS // 2 - 1)
        def _():
            g_issue(2 * g + 2, 0)

        g_wait(2 * g + 1, 1)

        @pl.when(g >= 1)
        def _():
            s_desc(2 * g - 1, 1).wait()

        compute(1)
        pltpu.async_copy(send[1],
                         acc_s.at[dst_t.at[pl.ds((2 * g + 1) * B, B)]],
                         ss[1], add=True)
        return carry

    lax.fori_loop(0, NCH_FS // 2, pair, 0)
    s_desc(NCH_FS - 2, 0).wait()
    s_desc(NCH_FS - 1, 1).wait()
    plsc.subcore_barrier()
    pltpu.sync_copy(acc_s.at[pl.ds(s * RPT, RPT)],
                    acc_out.at[pl.ds(c * NP + s * RPT, RPT)])


def _gat_call(gsrc_pad, dst_pad, htab, altabd, shift, zeros72):
    k = pl.kernel(
        _gat_body,
        out_type=jax.ShapeDtypeStruct((NC * NP, 72), _f32),
        mesh=_mesh(),
        scratch_types=[
            pltpu.VMEM((EPW_FS,), _i32),
            pltpu.VMEM((EPW_FS,), _i32),
            pltpu.VMEM((B, 80), _f32),
            pltpu.VMEM((B, 80), _f32),
            pltpu.VMEM((B, 16), _f32),
            pltpu.VMEM((B, 16), _f32),
            pltpu.VMEM((B, 72), _f32),
            pltpu.VMEM((B, 72), _f32),
            pltpu.VMEM((16,), _f32),
            pltpu.VMEM_SHARED((NP, 72), _f32),
            pltpu.SemaphoreType.DMA,
            pltpu.SemaphoreType.DMA,
            pltpu.SemaphoreType.DMA,
            pltpu.SemaphoreType.DMA,
        ],
        compiler_params=_SC_PARAMS,
    )
    return k(gsrc_pad, dst_pad, htab, altabd, shift, zeros72)


def _gat4_body(src_hbm, dst_hbm, htab4, altabd4, shift_hbm, zeros16, acc_out,
               src_t, dst_t, rows0, rows1, drow0, drow1, send0, send1,
               shift_v, acc_s, sg0, sg1, ss0, ss1):
    c = lax.axis_index("c")
    s = lax.axis_index("s")
    w = c * NS + s
    rows = (rows0, rows1)
    drow = (drow0, drow1)
    send = (send0, send1)
    sg = (sg0, sg1)
    ss = (ss0, ss1)
    for k in range(RPT // B):
        pltpu.sync_copy(zeros16, acc_s.at[pl.ds(s * RPT + k * B, B)])
    pltpu.sync_copy(src_hbm.at[pl.ds(w * EPW_ES, EPW_ES)], src_t)
    pltpu.sync_copy(dst_hbm.at[pl.ds(w * EPW_ES, EPW_ES)], dst_t)
    pltpu.sync_copy(shift_hbm, shift_v)
    plsc.subcore_barrier()
    shv = shift_v[...]
    lane3 = _bc16(3)

    def g_pair(ch, b):
        return (pltpu.make_async_copy(htab4.at[src_t.at[pl.ds(ch * B, B)]],
                                      rows[b], sg[b]),
                pltpu.make_async_copy(altabd4.at[dst_t.at[pl.ds(ch * B, B)]],
                                      drow[b], sg[b]))

    def g_issue(ch, b):
        for d in g_pair(ch, b):
            d.start()

    def g_wait(ch, b):
        for d in g_pair(ch, b):
            d.wait()

    def s_desc(ch, b):
        return pltpu.make_async_copy(
            send[b], acc_s.at[dst_t.at[pl.ds(ch * B, B)]], ss[b])

    def compute(b):
        for jj in range(B):
            sr = _lane_take(rows[b][jj, :], lane3)
            dr = drow[b][jj, :]
            e = sr + dr
            e = jnp.maximum(e, 0.2 * e)
            ex = jnp.exp(e - shv)
            send[b][jj, :] = rows[b][jj, :] * ex

    def step(ch, b, issue_next):
        if issue_next:
            g_issue(ch + 1, 1 - b)
        g_wait(ch, b)
        if isinstance(ch, int):
            if ch >= 2:
                s_desc(ch - 2, b).wait()
        else:
            @pl.when(ch >= 2)
            def _():
                s_desc(ch - 2, b).wait()

        compute(b)
        pltpu.async_copy(send[b], acc_s.at[dst_t.at[pl.ds(ch * B, B)]],
                         ss[b], add=True)

    g_issue(0, 0)
    NPAIR = NCH_ES // 2

    def pair(g, carry):
        step(2 * g, 0, True)
        step(2 * g + 1, 1, True)
        return carry

    lax.fori_loop(0, NPAIR, pair, 0)
    step(NCH_ES - 1, 0, False)
    s_desc(NCH_ES - 2, 1).wait()
    s_desc(NCH_ES - 1, 0).wait()
    plsc.subcore_barrier()
    pltpu.sync_copy(acc_s.at[pl.ds(s * RPT, RPT)],
                    acc_out.at[pl.ds(c * NP + s * RPT, RPT)])


def _gat4_call(src_pad, dst_pad, htab4, altabd4, shift4, zeros16):
    k = pl.kernel(
        _gat4_body,
        out_type=jax.ShapeDtypeStruct((NC * NP, 16), _f32),
        mesh=_mesh(),
        scratch_types=[
            pltpu.VMEM((EPW_ES,), _i32),
            pltpu.VMEM((EPW_ES,), _i32),
            pltpu.VMEM((B, 16), _f32),
            pltpu.VMEM((B, 16), _f32),
            pltpu.VMEM((B, 16), _f32),
            pltpu.VMEM((B, 16), _f32),
            pltpu.VMEM((B, 16), _f32),
            pltpu.VMEM((B, 16), _f32),
            pltpu.VMEM((16,), _f32),
            pltpu.VMEM_SHARED((NP, 16), _f32),
            pltpu.SemaphoreType.DMA,
            pltpu.SemaphoreType.DMA,
            pltpu.SemaphoreType.DMA,
            pltpu.SemaphoreType.DMA,
        ],
        compiler_params=_SC_PARAMS,
    )
    return k(src_pad, dst_pad, htab4, altabd4, shift4, zeros16)


def _elu(x):
    return jnp.where(x > 0, x, jnp.exp(jnp.minimum(x, 0.0)) - 1.0)


def _leaky(x):
    return jnp.maximum(x, 0.2 * x)


R0 = 2000
R = 1280
R5 = 1000


def _tc0_body(x_ref, wg_ref, deg_ref, g_ref):
    degs = deg_ref[0, :, 0:1] + deg_ref[1, :, 0:1]
    dinv = lax.rsqrt(jnp.maximum(degs, 1.0))
    g_ref[...] = (x_ref[...] @ wg_ref[...]) * dinv


def _tc0_call(x, W_gcn, deg2):
    return pl.pallas_call(
        _tc0_body,
        grid=(N // R0,),
        in_specs=[
            pl.BlockSpec((R0, D), lambda i: (i, 0)),
            pl.BlockSpec((D, 16), lambda i: (0, 0)),
            pl.BlockSpec((2, R0, 8), lambda i: (0, i, 0)),
        ],
        out_specs=pl.BlockSpec((R0, 16), lambda i: (i, 0)),
        out_shape=jax.ShapeDtypeStruct((NP, 16), _f32),
    )(x, W_gcn, deg2)


def _attn_tail(i, h, As_ref, Ad_ref, htab_ref, altab_ref, shift_ref, mxs, mxd):
    als = h @ As_ref[...]
    ald = h @ Ad_ref[...]
    z8 = jnp.zeros((h.shape[0], 8), _f32)
    htab_ref[...] = jnp.stack(
        [jnp.concatenate([h[:, :64], als, z8], axis=1),
         jnp.concatenate([h[:, 64:], als, z8], axis=1)], axis=0)
    altab_ref[...] = jnp.concatenate([ald, ald], axis=1)

    @pl.when(i == 0)
    def _():
        mxs[...] = jnp.full((1, 8), -1e30, _f32)
        mxd[...] = jnp.full((1, 8), -1e30, _f32)

    rblk = als.shape[0]
    valid = (lax.broadcasted_iota(_i32, (rblk, 8), 0) + i * rblk) < N
    mxs[...] = jnp.maximum(mxs[...],
                           jnp.max(jnp.where(valid, als, -1e30), axis=0,
                                   keepdims=True))
    mxd[...] = jnp.maximum(mxd[...],
                           jnp.max(jnp.where(valid, ald, -1e30), axis=0,
                                   keepdims=True))
    sh = _leaky(mxs[...] + mxd[...])
    shift_ref[...] = jnp.concatenate([jnp.zeros((1, 8), _f32), sh], axis=1)


def _tc1_body(accg_ref, deg_ref, bg_ref, W_ref, As_ref, Ad_ref,
              htab_ref, altab_ref, shift_ref, mxs, mxd):
    i = pl.program_id(0)
    g = accg_ref[0] + accg_ref[1]
    degs = deg_ref[0, :, 0:1] + deg_ref[1, :, 0:1]
    dinv = lax.rsqrt(jnp.maximum(degs, 1.0))
    x1 = _elu(g * dinv + bg_ref[...])
    h = x1 @ W_ref[...]
    _attn_tail(i, h, As_ref, Ad_ref, htab_ref, altab_ref, shift_ref, mxs, mxd)


def _gat_outs():
    return dict(
        out_specs=[
            pl.BlockSpec((2, R, 80), lambda i: (0, i, 0)),
            pl.BlockSpec((R, 16), lambda i: (i, 0)),
            pl.BlockSpec((1, 16), lambda i: (0, 0)),
        ],
        out_shape=[
            jax.ShapeDtypeStruct((2, NP, 80), _f32),
            jax.ShapeDtypeStruct((NP, 16), _f32),
            jax.ShapeDtypeStruct((1, 16), _f32),
        ],
        scratch_shapes=[pltpu.VMEM((1, 8), _f32), pltpu.VMEM((1, 8), _f32)],
    )


def _tc1_call(accg2, deg2, b_gcn, W1, As1, Ad1):
    return pl.pallas_call(
        _tc1_body,
        grid=(NP // R,),
        in_specs=[
            pl.BlockSpec((2, R, 16), lambda i: (0, i, 0)),
            pl.BlockSpec((2, R, 8), lambda i: (0, i, 0)),
            pl.BlockSpec((1, 16), lambda i: (0, 0)),
            pl.BlockSpec((16, D), lambda i: (0, 0)),
            pl.BlockSpec((D, 8), lambda i: (0, 0)),
            pl.BlockSpec((D, 8), lambda i: (0, 0)),
        ],
        **_gat_outs(),
    )(accg2, deg2, b_gcn, W1, As1, Ad1)


def _xin_from_acc(acc_ref, b_ref, Rep4):
    xs = []
    for cc in range(2):
        f = acc_ref[cc, :, 0:64]
        den = acc_ref[cc, :, 64 + 4 * cc:68 + 4 * cc] @ Rep4
        xs.append(f / (den + 1e-16))
    return _elu(jnp.concatenate(xs, axis=1) + b_ref[...])


def _tcmid_body(rep_ref, acc_ref, b_ref, W_ref, As_ref, Ad_ref,
                htab_ref, altab_ref, shift_ref, mxs, mxd):
    i = pl.program_id(0)
    x = _xin_from_acc(acc_ref, b_ref, rep_ref[...])
    h = x @ W_ref[...]
    _attn_tail(i, h, As_ref, Ad_ref, htab_ref, altab_ref, shift_ref, mxs, mxd)


def _tcmid_call(rep4, acc2, b_prev, W, As, Ad):
    return pl.pallas_call(
        _tcmid_body,
        grid=(NP // R,),
        in_specs=[
            pl.BlockSpec((4, 64), lambda i: (0, 0)),
            pl.BlockSpec((2, R, 72), lambda i: (0, i, 0)),
            pl.BlockSpec((1, D), lambda i: (0, 0)),
            pl.BlockSpec((D, D), lambda i: (0, 0)),
            pl.BlockSpec((D, 8), lambda i: (0, 0)),
            pl.BlockSpec((D, 8), lambda i: (0, 0)),
        ],
        **_gat_outs(),
    )(rep4, acc2, b_prev, W, As, Ad)


def _tc4_body(rep_ref, acc_ref, b_ref, W_ref, As_ref, Ad_ref,
              htab_ref, altab_ref, shift_ref, mxs, mxd):
    i = pl.program_id(0)
    x = _xin_from_acc(acc_ref, b_ref, rep_ref[...])
    h4 = x @ W_ref[...]
    als = h4 @ As_ref[...]
    ald = h4 @ Ad_ref[...]
    rblk = h4.shape[0]
    htab_ref[...] = jnp.concatenate(
        [h4, jnp.ones((rblk, 1), _f32), als, jnp.zeros((rblk, 12), _f32)],
        axis=1)
    altab_ref[...] = jnp.concatenate([ald] * 16, axis=1)

    @pl.when(i == 0)
    def _():
        mxs[...] = jnp.full((1, 8), -1e30, _f32)
        mxd[...] = jnp.full((1, 8), -1e30, _f32)

    valid = (lax.broadcasted_iota(_i32, (rblk, 1), 0) + i * rblk) < N
    mxs[...] = jnp.maximum(
        mxs[...],
        jnp.max(jnp.where(valid, als, -1e30), axis=0, keepdims=True))
    mxd[...] = jnp.maximum(
        mxd[...],
        jnp.max(jnp.where(valid, ald, -1e30), axis=0, keepdims=True))
    sh = _leaky(mxs[...] + mxd[...])
    shift_ref[...] = jnp.concatenate(
        [sh[:, 0:1], jnp.zeros((1, 15), _f32)], axis=1)


def _tc4_call(rep4, acc2, b3, W4, As4, Ad4):
    return pl.pallas_call(
        _tc4_body,
        grid=(NP // R,),
        in_specs=[
            pl.BlockSpec((4, 64), lambda i: (0, 0)),
            pl.BlockSpec((2, R, 72), lambda i: (0, i, 0)),
            pl.BlockSpec((1, D), lambda i: (0, 0)),
            pl.BlockSpec((D, 2), lambda i: (0, 0)),
            pl.BlockSpec((2, 1), lambda i: (0, 0)),
            pl.BlockSpec((2, 1), lambda i: (0, 0)),
        ],
        out_specs=[
            pl.BlockSpec((R, 16), lambda i: (i, 0)),
            pl.BlockSpec((R, 16), lambda i: (i, 0)),
            pl.BlockSpec((1, 16), lambda i: (0, 0)),
        ],
        out_shape=[
            jax.ShapeDtypeStruct((NP, 16), _f32),
            jax.ShapeDtypeStruct((NP, 16), _f32),
            jax.ShapeDtypeStruct((1, 16), _f32),
        ],
        scratch_shapes=[pltpu.VMEM((1, 8), _f32), pltpu.VMEM((1, 8), _f32)],
    )(rep4, acc2, b3, W4, As4, Ad4)


def _tc5_body(accU_ref, accI_ref, b_ref, out_ref):
    def node_h(a):
        f = a[0, :, 0:2] + a[1, :, 0:2]
        den = a[0, :, 2:3] + a[1, :, 2:3]
        return _elu(f / (den + 1e-16) + b_ref[...])

    z = jnp.concatenate([node_h(accU_ref[...]), node_h(accI_ref[...])], axis=1)
    m = jnp.max(z, axis=1, keepdims=True)
    lse = jnp.log(jnp.sum(jnp.exp(z - m), axis=1, keepdims=True)) + m
    out_ref[...] = z - lse


def _tc5_call(acc42, b4):
    return pl.pallas_call(
        _tc5_body,
        grid=(N // 2 // R5,),
        in_specs=[
            pl.BlockSpec((2, R5, 16), lambda i: (0, i, 0)),
            pl.BlockSpec((2, R5, 16), lambda i: (0, i + 5, 0)),
            pl.BlockSpec((1, 2), lambda i: (0, 0)),
        ],
        out_specs=pl.BlockSpec((R5, 4), lambda i: (i, 0)),
        out_shape=jax.ShapeDtypeStruct((N // 2, 4), _f32),
    )(acc42, acc42, b4)


def _head_proj(a):
    H, C = a.shape
    m = jnp.zeros((H * C, H), _f32)
    hh = jnp.arange(H * C) // C
    return m.at[jnp.arange(H * C), hh].set(a.reshape(-1))


def kernel(x, edge_index, batch, W_gcn, b_gcn, W1, as1, ad1, b1,
           W2, as2, ad2, b2, W3, as3, ad3, b3, W4, as4, ad4, b4):
    ar = jnp.arange(N, dtype=_i32)
    npad = ET - E - N
    src_pad = jnp.concatenate(
        [edge_index[0], ar, jnp.full((npad,), N, _i32)])
    dst_pad = jnp.concatenate(
        [edge_index[1], ar, jnp.full((npad,), N, _i32)])

    zeros8 = jnp.zeros((B, 8), _f32)
    ones8 = zeros8.at[:, 0].set(1.0)
    zeros16 = jnp.zeros((B, 16), _f32)
    zeros72 = jnp.zeros((B, 72), _f32)
    rep4 = jnp.repeat(jnp.eye(4, dtype=_f32), 16, axis=1)

    As1, Ad1 = _head_proj(as1), _head_proj(ad1)
    As2, Ad2 = _head_proj(as2), _head_proj(ad2)
    As3, Ad3 = _head_proj(as3), _head_proj(ad3)
    As4, Ad4 = _head_proj(as4), _head_proj(ad4)

    deg2 = _deg_call(dst_pad, zeros8, ones8).reshape(2, NP, 8)
    gtab = _tc0_call(x, W_gcn, deg2)
    accg = _gcn_call(src_pad, dst_pad, gtab, zeros16).reshape(2, NP, 16)

    gsrc_pad = jnp.concatenate([src_pad, src_pad + NP])

    htab, altab, shift = _tc1_call(accg, deg2, b_gcn.reshape(1, 16),
                                   W1, As1, Ad1)
    acc1 = _gat_call(gsrc_pad, dst_pad, htab.reshape(2 * NP, 80),
                     altab, shift.reshape(16), zeros72)

    htab, altab, shift = _tcmid_call(rep4, acc1.reshape(2, NP, 72),
                                     b1.reshape(1, D), W2, As2, Ad2)
    acc2 = _gat_call(gsrc_pad, dst_pad, htab.reshape(2 * NP, 80),
                     altab, shift.reshape(16), zeros72)
    htab, altab, shift = _tcmid_call(rep4, acc2.reshape(2, NP, 72),
                                     b2.reshape(1, D), W3, As3, Ad3)
    acc3 = _gat_call(gsrc_pad, dst_pad, htab.reshape(2 * NP, 80),
                     altab, shift.reshape(16), zeros72)

    htab4, altab4, shift4 = _tc4_call(rep4, acc3.reshape(2, NP, 72),
                                      b3.reshape(1, D), W4, As4, Ad4)
    acc4 = _gat4_call(src_pad, dst_pad, htab4, altab4,
                      shift4.reshape(16), zeros16)

    return _tc5_call(acc4.reshape(2, NP, 16), b4.reshape(1, 2))


# speedup vs baseline: 1.0529x; 1.0529x over previous
import jax
import jax.numpy as jnp
from jax import lax
from jax.experimental import pallas as pl
from jax.experimental.pallas import tpu as pltpu
from jax.experimental.pallas import tpu_sc as plsc

N = 10000
D = 128
E = 320000
NC, NS, L = 2, 16, 16
NW = NC * NS
NP = 10240
RPT = NP // NS
B = 128
ET = 331776
EPW_ES = ET // NW
EPW_FS = ET // NS
NCH_ES = EPW_ES // B
NCH_FS = EPW_FS // B

_f32 = jnp.float32
_i32 = jnp.int32

_SC_PARAMS = pltpu.CompilerParams(use_tc_tiling_on_sc=False)


def _mesh():
    return plsc.VectorSubcoreMesh(core_axis_name="c", subcore_axis_name="s",
                                  num_cores=NC, num_subcores=NS)


def _lane_take(x, idx):
    dn = lax.GatherDimensionNumbers(offset_dims=(), collapsed_slice_dims=(0,),
                                    start_index_map=(0,))
    return lax.gather(x, idx[:, None], dn, (1,),
                      mode=lax.GatherScatterMode.PROMISE_IN_BOUNDS)


def _bc16(v):
    return jnp.zeros((16,), _i32) + v


def _deg_body(dst_hbm, zeros8, ones8, deg_out, dst_v, row_v, acc_s, sem):
    c = lax.axis_index("c")
    s = lax.axis_index("s")
    w = c * NS + s
    for k in range(RPT // B):
        pltpu.sync_copy(zeros8, acc_s.at[pl.ds(s * RPT + k * B, B)])
    plsc.subcore_barrier()
    pltpu.sync_copy(ones8, row_v)

    def chunk(i, carry):
        base = w * EPW_ES + i * B
        pltpu.sync_copy(dst_hbm.at[pl.ds(base, B)], dst_v)
        pltpu.sync_copy(row_v, acc_s.at[dst_v], add=True)
        return carry

    lax.fori_loop(0, NCH_ES, chunk, 0)
    plsc.subcore_barrier()
    pltpu.sync_copy(acc_s.at[pl.ds(s * RPT, RPT)],
                    deg_out.at[pl.ds(c * NP + s * RPT, RPT)])


def _deg_call(dst_pad, zeros8, ones8):
    k = pl.kernel(
        _deg_body,
        out_type=jax.ShapeDtypeStruct((NC * NP, 8), _f32),
        mesh=_mesh(),
        scratch_types=[
            pltpu.VMEM((B,), _i32),
            pltpu.VMEM((B, 8), _f32),
            pltpu.VMEM_SHARED((NP, 8), _f32),
            pltpu.SemaphoreType.DMA,
        ],
        compiler_params=_SC_PARAMS,
    )
    return k(dst_pad, zeros8, ones8)


def _gcn_body(src_hbm, dst_hbm, gtab, zeros16, acc_out,
              src_t, dst_t, rows0, rows1, acc_s, sg0, sg1, ss0, ss1):
    c = lax.axis_index("c")
    s = lax.axis_index("s")
    w = c * NS + s
    rows = (rows0, rows1)
    sg = (sg0, sg1)
    ss = (ss0, ss1)
    for k in range(RPT // B):
        pltpu.sync_copy(zeros16, acc_s.at[pl.ds(s * RPT + k * B, B)])
    pltpu.sync_copy(src_hbm.at[pl.ds(w * EPW_ES, EPW_ES)], src_t)
    pltpu.sync_copy(dst_hbm.at[pl.ds(w * EPW_ES, EPW_ES)], dst_t)
    plsc.subcore_barrier()

    def g_desc(ch, b):
        return pltpu.make_async_copy(gtab.at[src_t.at[pl.ds(ch * B, B)]],
                                     rows[b], sg[b])

    def s_desc(ch, b):
        return pltpu.make_async_copy(
            rows[b], acc_s.at[dst_t.at[pl.ds(ch * B, B)]], ss[b])

    def step(ch, b, issue_next, first=False):
        if not first:
            s_desc(ch - 1, 1 - b).wait()
        if issue_next:
            g_desc(ch + 1, 1 - b).start()
        g_desc(ch, b).wait()
        pltpu.async_copy(rows[b], acc_s.at[dst_t.at[pl.ds(ch * B, B)]],
                         ss[b], add=True)

    g_desc(0, 0).start()

    def pair(g, carry):
        @pl.when(g == 0)
        def _():
            g_desc(1, 1).start()
            g_desc(0, 0).wait()
            pltpu.async_copy(rows[0], acc_s.at[dst_t.at[pl.ds(0, B)]],
                             ss[0], add=True)

        @pl.when(g > 0)
        def _():
            step(2 * g, 0, True)

        step(2 * g + 1, 1, True)
        return carry

    lax.fori_loop(0, NCH_ES // 2, pair, 0)
    step(NCH_ES - 1, 0, False)
    s_desc(NCH_ES - 1, 0).wait()
    plsc.subcore_barrier()
    pltpu.sync_copy(acc_s.at[pl.ds(s * RPT, RPT)],
                    acc_out.at[pl.ds(c * NP + s * RPT, RPT)])


def _gcn_call(src_pad, dst_pad, gtab, zeros16):
    k = pl.kernel(
        _gcn_body,
        out_type=jax.ShapeDtypeStruct((NC * NP, 16), _f32),
        mesh=_mesh(),
        scratch_types=[
            pltpu.VMEM((EPW_ES,), _i32),
            pltpu.VMEM((EPW_ES,), _i32),
            pltpu.VMEM((B, 16), _f32),
            pltpu.VMEM((B, 16), _f32),
            pltpu.VMEM_SHARED((NP, 16), _f32),
            pltpu.SemaphoreType.DMA,
            pltpu.SemaphoreType.DMA,
            pltpu.SemaphoreType.DMA,
            pltpu.SemaphoreType.DMA,
        ],
        compiler_params=_SC_PARAMS,
    )
    return k(src_pad, dst_pad, gtab, zeros16)


def _gat_body(gsrc_hbm, dst_hbm, htab, altabd, shift_hbm, zeros72, acc_out,
              gidx_t, dst_t, rows0, rows1, drow0, drow1, send0, send1,
              shift_v, acc_s, sg0, sg1, ss0, ss1):
    c = lax.axis_index("c")
    s = lax.axis_index("s")
    rows = (rows0, rows1)
    drow = (drow0, drow1)
    send = (send0, send1)
    sg = (sg0, sg1)
    ss = (ss0, ss1)
    for k in range(RPT // B):
        pltpu.sync_copy(zeros72, acc_s.at[pl.ds(s * RPT + k * B, B)])
    pltpu.sync_copy(gsrc_hbm.at[pl.ds(c * ET + s * EPW_FS, EPW_FS)], gidx_t)
    pltpu.sync_copy(dst_hbm.at[pl.ds(s * EPW_FS, EPW_FS)], dst_t)
    pltpu.sync_copy(shift_hbm, shift_v)
    plsc.subcore_barrier()
    shv = shift_v[...]

    def g_pair(ch, b):
        return (pltpu.make_async_copy(htab.at[gidx_t.at[pl.ds(ch * B, B)]],
                                      rows[b], sg[b]),
                pltpu.make_async_copy(altabd.at[dst_t.at[pl.ds(ch * B, B)]],
                                      drow[b], sg[b]))

    def g_issue(ch, b):
        for d in g_pair(ch, b):
            d.start()

    def g_wait(ch, b):
        for d in g_pair(ch, b):
            d.wait()

    def s_desc(ch, b):
        return pltpu.make_async_copy(
            send[b], acc_s.at[dst_t.at[pl.ds(ch * B, B)]], ss[b])

    def compute(b):
        for jj in range(B):
            sr = rows[b][jj, pl.ds(56, 16)]
            dr = drow[b][jj, :]
            e = sr + dr
            e = jnp.maximum(e, 0.2 * e)
            ex = jnp.exp(e - shv)
            send[b][jj, pl.ds(56, 16)] = ex
            for h in range(4):
                exb = _lane_take(ex, _bc16(8 + 4 * c + h))
                send[b][jj, pl.ds(16 * h, 16)] = (
                    rows[b][jj, pl.ds(16 * h, 16)] * exb)

    g_issue(0, 0)

    def pair(g, carry):
        g_issue(2 * g + 1, 1)
        g_wait(2 * g, 0)

        @pl.when(g >= 1)
        def _():
            s_desc(2 * g - 2, 0).wait()

        compute(0)
        pltpu.async_copy(send[0], acc_s.at[dst_t.at[pl.ds((2 * g) * B, B)]],
                         ss[0], add=True)
        @pl.when(g < NCH_FS // 2 - 1)
        def _():
            g_issue(2 * g + 2, 0)

        g_wait(2 * g + 1, 1)

        @pl.when(g >= 1)
        def _():
            s_desc(2 * g - 1, 1).wait()

        compute(1)
        pltpu.async_copy(send[1],
                         acc_s.at[dst_t.at[pl.ds((2 * g + 1) * B, B)]],
                         ss[1], add=True)
        return carry

    lax.fori_loop(0, NCH_FS // 2, pair, 0)
    s_desc(NCH_FS - 2, 0).wait()
    s_desc(NCH_FS - 1, 1).wait()
    plsc.subcore_barrier()
    pltpu.sync_copy(acc_s.at[pl.ds(s * RPT, RPT)],
                    acc_out.at[pl.ds(c * NP + s * RPT, RPT)])


def _gat_call(gsrc_pad, dst_pad, htab, altabd, shift, zeros72):
    k = pl.kernel(
        _gat_body,
        out_type=jax.ShapeDtypeStruct((NC * NP, 72), _f32),
        mesh=_mesh(),
        scratch_types=[
            pltpu.VMEM((EPW_FS,), _i32),
            pltpu.VMEM((EPW_FS,), _i32),
            pltpu.VMEM((B, 72), _f32),
            pltpu.VMEM((B, 72), _f32),
            pltpu.VMEM((B, 16), _f32),
            pltpu.VMEM((B, 16), _f32),
            pltpu.VMEM((B, 72), _f32),
            pltpu.VMEM((B, 72), _f32),
            pltpu.VMEM((16,), _f32),
            pltpu.VMEM_SHARED((NP, 72), _f32),
            pltpu.SemaphoreType.DMA,
            pltpu.SemaphoreType.DMA,
            pltpu.SemaphoreType.DMA,
            pltpu.SemaphoreType.DMA,
        ],
        compiler_params=_SC_PARAMS,
    )
    return k(gsrc_pad, dst_pad, htab, altabd, shift, zeros72)


def _gat4_body(src_hbm, dst_hbm, htab4, altabd4, shift_hbm, zeros16, acc_out,
               src_t, dst_t, rows0, rows1, drow0, drow1, send0, send1,
               shift_v, acc_s, sg0, sg1, ss0, ss1):
    c = lax.axis_index("c")
    s = lax.axis_index("s")
    w = c * NS + s
    rows = (rows0, rows1)
    drow = (drow0, drow1)
    send = (send0, send1)
    sg = (sg0, sg1)
    ss = (ss0, ss1)
    for k in range(RPT // B):
        pltpu.sync_copy(zeros16, acc_s.at[pl.ds(s * RPT + k * B, B)])
    pltpu.sync_copy(src_hbm.at[pl.ds(w * EPW_ES, EPW_ES)], src_t)
    pltpu.sync_copy(dst_hbm.at[pl.ds(w * EPW_ES, EPW_ES)], dst_t)
    pltpu.sync_copy(shift_hbm, shift_v)
    plsc.subcore_barrier()
    shv = shift_v[...]
    lane3 = _bc16(3)

    def g_pair(ch, b):
        return (pltpu.make_async_copy(htab4.at[src_t.at[pl.ds(ch * B, B)]],
                                      rows[b], sg[b]),
                pltpu.make_async_copy(altabd4.at[dst_t.at[pl.ds(ch * B, B)]],
                                      drow[b], sg[b]))

    def g_issue(ch, b):
        for d in g_pair(ch, b):
            d.start()

    def g_wait(ch, b):
        for d in g_pair(ch, b):
            d.wait()

    def s_desc(ch, b):
        return pltpu.make_async_copy(
            send[b], acc_s.at[dst_t.at[pl.ds(ch * B, B)]], ss[b])

    def compute(b):
        for jj in range(B):
            sr = _lane_take(rows[b][jj, :], lane3)
            dr = drow[b][jj, :]
            e = sr + dr
            e = jnp.maximum(e, 0.2 * e)
            ex = jnp.exp(e - shv)
            send[b][jj, :] = rows[b][jj, :] * ex

    def step(ch, b, issue_next):
        if issue_next:
            g_issue(ch + 1, 1 - b)
        g_wait(ch, b)
        if isinstance(ch, int):
            if ch >= 2:
                s_desc(ch - 2, b).wait()
        else:
            @pl.when(ch >= 2)
            def _():
                s_desc(ch - 2, b).wait()

        compute(b)
        pltpu.async_copy(send[b], acc_s.at[dst_t.at[pl.ds(ch * B, B)]],
                         ss[b], add=True)

    g_issue(0, 0)
    NPAIR = NCH_ES // 2

    def pair(g, carry):
        step(2 * g, 0, True)
        step(2 * g + 1, 1, True)
        return carry

    lax.fori_loop(0, NPAIR, pair, 0)
    step(NCH_ES - 1, 0, False)
    s_desc(NCH_ES - 2, 1).wait()
    s_desc(NCH_ES - 1, 0).wait()
    plsc.subcore_barrier()
    pltpu.sync_copy(acc_s.at[pl.ds(s * RPT, RPT)],
                    acc_out.at[pl.ds(c * NP + s * RPT, RPT)])


def _gat4_call(src_pad, dst_pad, htab4, altabd4, shift4, zeros16):
    k = pl.kernel(
        _gat4_body,
        out_type=jax.ShapeDtypeStruct((NC * NP, 16), _f32),
        mesh=_mesh(),
        scratch_types=[
            pltpu.VMEM((EPW_ES,), _i32),
            pltpu.VMEM((EPW_ES,), _i32),
            pltpu.VMEM((B, 16), _f32),
            pltpu.VMEM((B, 16), _f32),
            pltpu.VMEM((B, 16), _f32),
            pltpu.VMEM((B, 16), _f32),
            pltpu.VMEM((B, 16), _f32),
            pltpu.VMEM((B, 16), _f32),
            pltpu.VMEM((16,), _f32),
            pltpu.VMEM_SHARED((NP, 16), _f32),
            pltpu.SemaphoreType.DMA,
            pltpu.SemaphoreType.DMA,
            pltpu.SemaphoreType.DMA,
            pltpu.SemaphoreType.DMA,
        ],
        compiler_params=_SC_PARAMS,
    )
    return k(src_pad, dst_pad, htab4, altabd4, shift4, zeros16)


def _elu(x):
    return jnp.where(x > 0, x, jnp.exp(jnp.minimum(x, 0.0)) - 1.0)


def _leaky(x):
    return jnp.maximum(x, 0.2 * x)


R0 = 2000
R = 1280
R5 = 1000


def _tc0a_body(x_ref, wg_ref, h0_ref):
    h0_ref[...] = x_ref[...] @ wg_ref[...]


def _tc0a_call(x, W_gcn):
    return pl.pallas_call(
        _tc0a_body,
        grid=(N // R0,),
        in_specs=[
            pl.BlockSpec((R0, D), lambda i: (i, 0)),
            pl.BlockSpec((D, 16), lambda i: (0, 0)),
        ],
        out_specs=pl.BlockSpec((R0, 16), lambda i: (i, 0)),
        out_shape=jax.ShapeDtypeStruct((NP, 16), _f32),
    )(x, W_gcn)


def _tc0b_body(h0_ref, deg_ref, g_ref):
    degs = deg_ref[0, :, 0:1] + deg_ref[1, :, 0:1]
    dinv = lax.rsqrt(jnp.maximum(degs, 1.0))
    g_ref[...] = h0_ref[...] * dinv


def _tc0b_call(h0, deg2):
    return pl.pallas_call(
        _tc0b_body,
        grid=(N // R0,),
        in_specs=[
            pl.BlockSpec((R0, 16), lambda i: (i, 0)),
            pl.BlockSpec((2, R0, 8), lambda i: (0, i, 0)),
        ],
        out_specs=pl.BlockSpec((R0, 16), lambda i: (i, 0)),
        out_shape=jax.ShapeDtypeStruct((NP, 16), _f32),
    )(h0, deg2)


def _attn_tail(i, h, As_ref, Ad_ref, htab_ref, altab_ref, shift_ref, mxs, mxd):
    als = h @ As_ref[...]
    ald = h @ Ad_ref[...]
    htab_ref[...] = jnp.stack(
        [jnp.concatenate([h[:, :64], als], axis=1),
         jnp.concatenate([h[:, 64:], als], axis=1)], axis=0)
    altab_ref[...] = jnp.concatenate([ald, ald], axis=1)

    @pl.when(i == 0)
    def _():
        mxs[...] = jnp.full((1, 8), -1e30, _f32)
        mxd[...] = jnp.full((1, 8), -1e30, _f32)

    rblk = als.shape[0]
    valid = (lax.broadcasted_iota(_i32, (rblk, 8), 0) + i * rblk) < N
    mxs[...] = jnp.maximum(mxs[...],
                           jnp.max(jnp.where(valid, als, -1e30), axis=0,
                                   keepdims=True))
    mxd[...] = jnp.maximum(mxd[...],
                           jnp.max(jnp.where(valid, ald, -1e30), axis=0,
                                   keepdims=True))
    sh = _leaky(mxs[...] + mxd[...])
    shift_ref[...] = jnp.concatenate([jnp.zeros((1, 8), _f32), sh], axis=1)


def _tc1_body(accg_ref, deg_ref, bg_ref, W_ref, As_ref, Ad_ref,
              htab_ref, altab_ref, shift_ref, mxs, mxd):
    i = pl.program_id(0)
    g = accg_ref[0] + accg_ref[1]
    degs = deg_ref[0, :, 0:1] + deg_ref[1, :, 0:1]
    dinv = lax.rsqrt(jnp.maximum(degs, 1.0))
    x1 = _elu(g * dinv + bg_ref[...])
    h = x1 @ W_ref[...]
    _attn_tail(i, h, As_ref, Ad_ref, htab_ref, altab_ref, shift_ref, mxs, mxd)


def _gat_outs():
    return dict(
        out_specs=[
            pl.BlockSpec((2, R, 72), lambda i: (0, i, 0)),
            pl.BlockSpec((R, 16), lambda i: (i, 0)),
            pl.BlockSpec((1, 16), lambda i: (0, 0)),
        ],
        out_shape=[
            jax.ShapeDtypeStruct((2, NP, 72), _f32),
            jax.ShapeDtypeStruct((NP, 16), _f32),
            jax.ShapeDtypeStruct((1, 16), _f32),
        ],
        scratch_shapes=[pltpu.VMEM((1, 8), _f32), pltpu.VMEM((1, 8), _f32)],
    )


def _tc1_call(accg2, deg2, b_gcn, W1, As1, Ad1):
    return pl.pallas_call(
        _tc1_body,
        grid=(NP // R,),
        in_specs=[
            pl.BlockSpec((2, R, 16), lambda i: (0, i, 0)),
            pl.BlockSpec((2, R, 8), lambda i: (0, i, 0)),
            pl.BlockSpec((1, 16), lambda i: (0, 0)),
            pl.BlockSpec((16, D), lambda i: (0, 0)),
            pl.BlockSpec((D, 8), lambda i: (0, 0)),
            pl.BlockSpec((D, 8), lambda i: (0, 0)),
        ],
        **_gat_outs(),
    )(accg2, deg2, b_gcn, W1, As1, Ad1)


def _xin_from_acc(acc_ref, b_ref, Rep4):
    xs = []
    for cc in range(2):
        f = acc_ref[cc, :, 0:64]
        den = acc_ref[cc, :, 64 + 4 * cc:68 + 4 * cc] @ Rep4
        xs.append(f / (den + 1e-16))
    return _elu(jnp.concatenate(xs, axis=1) + b_ref[...])


def _tcmid_body(rep_ref, acc_ref, b_ref, W_ref, As_ref, Ad_ref,
                htab_ref, altab_ref, shift_ref, mxs, mxd):
    i = pl.program_id(0)
    x = _xin_from_acc(acc_ref, b_ref, rep_ref[...])
    h = x @ W_ref[...]
    _attn_tail(i, h, As_ref, Ad_ref, htab_ref, altab_ref, shift_ref, mxs, mxd)


def _tcmid_call(rep4, acc2, b_prev, W, As, Ad):
    return pl.pallas_call(
        _tcmid_body,
        grid=(NP // R,),
        in_specs=[
            pl.BlockSpec((4, 64), lambda i: (0, 0)),
            pl.BlockSpec((2, R, 72), lambda i: (0, i, 0)),
            pl.BlockSpec((1, D), lambda i: (0, 0)),
            pl.BlockSpec((D, D), lambda i: (0, 0)),
            pl.BlockSpec((D, 8), lambda i: (0, 0)),
            pl.BlockSpec((D, 8), lambda i: (0, 0)),
        ],
        **_gat_outs(),
    )(rep4, acc2, b_prev, W, As, Ad)


def _tc4_body(rep_ref, acc_ref, b_ref, W_ref, As_ref, Ad_ref,
              htab_ref, altab_ref, shift_ref, mxs, mxd):
    i = pl.program_id(0)
    x = _xin_from_acc(acc_ref, b_ref, rep_ref[...])
    h4 = x @ W_ref[...]
    als = h4 @ As_ref[...]
    ald = h4 @ Ad_ref[...]
    rblk = h4.shape[0]
    htab_ref[...] = jnp.concatenate(
        [h4, jnp.ones((rblk, 1), _f32), als, jnp.zeros((rblk, 12), _f32)],
        axis=1)
    altab_ref[...] = jnp.concatenate([ald] * 16, axis=1)

    @pl.when(i == 0)
    def _():
        mxs[...] = jnp.full((1, 8), -1e30, _f32)
        mxd[...] = jnp.full((1, 8), -1e30, _f32)

    valid = (lax.broadcasted_iota(_i32, (rblk, 1), 0) + i * rblk) < N
    mxs[...] = jnp.maximum(
        mxs[...],
        jnp.max(jnp.where(valid, als, -1e30), axis=0, keepdims=True))
    mxd[...] = jnp.maximum(
        mxd[...],
        jnp.max(jnp.where(valid, ald, -1e30), axis=0, keepdims=True))
    sh = _leaky(mxs[...] + mxd[...])
    shift_ref[...] = jnp.concatenate(
        [sh[:, 0:1], jnp.zeros((1, 15), _f32)], axis=1)


def _tc4_call(rep4, acc2, b3, W4, As4, Ad4):
    return pl.pallas_call(
        _tc4_body,
        grid=(NP // R,),
        in_specs=[
            pl.BlockSpec((4, 64), lambda i: (0, 0)),
            pl.BlockSpec((2, R, 72), lambda i: (0, i, 0)),
            pl.BlockSpec((1, D), lambda i: (0, 0)),
            pl.BlockSpec((D, 2), lambda i: (0, 0)),
            pl.BlockSpec((2, 1), lambda i: (0, 0)),
            pl.BlockSpec((2, 1), lambda i: (0, 0)),
        ],
        out_specs=[
            pl.BlockSpec((R, 16), lambda i: (i, 0)),
            pl.BlockSpec((R, 16), lambda i: (i, 0)),
            pl.BlockSpec((1, 16), lambda i: (0, 0)),
        ],
        out_shape=[
            jax.ShapeDtypeStruct((NP, 16), _f32),
            jax.ShapeDtypeStruct((NP, 16), _f32),
            jax.ShapeDtypeStruct((1, 16), _f32),
        ],
        scratch_shapes=[pltpu.VMEM((1, 8), _f32), pltpu.VMEM((1, 8), _f32)],
    )(rep4, acc2, b3, W4, As4, Ad4)


def _tc5_body(accU_ref, accI_ref, b_ref, out_ref):
    def node_h(a):
        f = a[0, :, 0:2] + a[1, :, 0:2]
        den = a[0, :, 2:3] + a[1, :, 2:3]
        return _elu(f / (den + 1e-16) + b_ref[...])

    z = jnp.concatenate([node_h(accU_ref[...]), node_h(accI_ref[...])], axis=1)
    m = jnp.max(z, axis=1, keepdims=True)
    lse = jnp.log(jnp.sum(jnp.exp(z - m), axis=1, keepdims=True)) + m
    out_ref[...] = z - lse


def _tc5_call(acc42, b4):
    return pl.pallas_call(
        _tc5_body,
        grid=(N // 2 // R5,),
        in_specs=[
            pl.BlockSpec((2, R5, 16), lambda i: (0, i, 0)),
            pl.BlockSpec((2, R5, 16), lambda i: (0, i + 5, 0)),
            pl.BlockSpec((1, 2), lambda i: (0, 0)),
        ],
        out_specs=pl.BlockSpec((R5, 4), lambda i: (i, 0)),
        out_shape=jax.ShapeDtypeStruct((N // 2, 4), _f32),
    )(acc42, acc42, b4)


def _head_proj(a):
    H, C = a.shape
    m = jnp.zeros((H * C, H), _f32)
    hh = jnp.arange(H * C) // C
    return m.at[jnp.arange(H * C), hh].set(a.reshape(-1))


def kernel(x, edge_index, batch, W_gcn, b_gcn, W1, as1, ad1, b1,
           W2, as2, ad2, b2, W3, as3, ad3, b3, W4, as4, ad4, b4):
    ar = jnp.arange(N, dtype=_i32)
    npad = ET - E - N
    src_pad = jnp.concatenate(
        [edge_index[0], ar, jnp.full((npad,), N, _i32)])
    dst_pad = jnp.concatenate(
        [edge_index[1], ar, jnp.full((npad,), N, _i32)])

    zeros8 = jnp.zeros((B, 8), _f32)
    ones8 = zeros8.at[:, 0].set(1.0)
    zeros16 = jnp.zeros((B, 16), _f32)
    zeros72 = jnp.zeros((B, 72), _f32)
    rep4 = jnp.repeat(jnp.eye(4, dtype=_f32), 16, axis=1)

    As1, Ad1 = _head_proj(as1), _head_proj(ad1)
    As2, Ad2 = _head_proj(as2), _head_proj(ad2)
    As3, Ad3 = _head_proj(as3), _head_proj(ad3)
    As4, Ad4 = _head_proj(as4), _head_proj(ad4)

    h0 = _tc0a_call(x, W_gcn)
    deg2 = _deg_call(dst_pad, zeros8, ones8).reshape(2, NP, 8)
    gtab = _tc0b_call(h0, deg2)
    accg = _gcn_call(src_pad, dst_pad, gtab, zeros16).reshape(2, NP, 16)

    gsrc_pad = jnp.concatenate([src_pad, src_pad + NP])

    htab, altab, shift = _tc1_call(accg, deg2, b_gcn.reshape(1, 16),
                                   W1, As1, Ad1)
    acc1 = _gat_call(gsrc_pad, dst_pad, htab.reshape(2 * NP, 72),
                     altab, shift.reshape(16), zeros72)

    htab, altab, shift = _tcmid_call(rep4, acc1.reshape(2, NP, 72),
                                     b1.reshape(1, D), W2, As2, Ad2)
    acc2 = _gat_call(gsrc_pad, dst_pad, htab.reshape(2 * NP, 72),
                     altab, shift.reshape(16), zeros72)
    htab, altab, shift = _tcmid_call(rep4, acc2.reshape(2, NP, 72),
                                     b2.reshape(1, D), W3, As3, Ad3)
    acc3 = _gat_call(gsrc_pad, dst_pad, htab.reshape(2 * NP, 72),
                     altab, shift.reshape(16), zeros72)

    htab4, altab4, shift4 = _tc4_call(rep4, acc3.reshape(2, NP, 72),
                                      b3.reshape(1, D), W4, As4, Ad4)
    acc4 = _gat4_call(src_pad, dst_pad, htab4, altab4,
                      shift4.reshape(16), zeros16)

    return _tc5_call(acc4.reshape(2, NP, 16), b4.reshape(1, 2))


# speedup vs baseline: 1.0885x; 1.0338x over previous
import jax
import jax.numpy as jnp
from jax import lax
from jax.experimental import pallas as pl
from jax.experimental.pallas import tpu as pltpu
from jax.experimental.pallas import tpu_sc as plsc

N = 10000
D = 128
E = 320000
NC, NS, L = 2, 16, 16
NW = NC * NS
NP = 10240
RPT = NP // NS
B = 128
ET = 331776
EPW_ES = ET // NW
EPW_FS = ET // NS
NCH_ES = EPW_ES // B
NCH_FS = EPW_FS // B

_f32 = jnp.float32
_i32 = jnp.int32

_SC_PARAMS = pltpu.CompilerParams(use_tc_tiling_on_sc=False)


def _mesh():
    return plsc.VectorSubcoreMesh(core_axis_name="c", subcore_axis_name="s",
                                  num_cores=NC, num_subcores=NS)


def _lane_take(x, idx):
    dn = lax.GatherDimensionNumbers(offset_dims=(), collapsed_slice_dims=(0,),
                                    start_index_map=(0,))
    return lax.gather(x, idx[:, None], dn, (1,),
                      mode=lax.GatherScatterMode.PROMISE_IN_BOUNDS)


def _bc16(v):
    return jnp.zeros((16,), _i32) + v


def _deg_body(dst_hbm, zeros8, ones8, deg_out, dst_t, row_v, acc_s, sem):
    c = lax.axis_index("c")
    s = lax.axis_index("s")
    w = c * NS + s
    for k in range(RPT // B):
        pltpu.sync_copy(zeros8, acc_s.at[pl.ds(s * RPT + k * B, B)])
    pltpu.sync_copy(dst_hbm.at[pl.ds(w * EPW_ES, EPW_ES)], dst_t)
    pltpu.sync_copy(ones8, row_v)
    plsc.subcore_barrier()

    def s_desc(ch):
        return pltpu.make_async_copy(
            row_v, acc_s.at[dst_t.at[pl.ds(ch * B, B)]], sem)

    K = 3
    for ch in range(K):
        pltpu.async_copy(row_v, acc_s.at[dst_t.at[pl.ds(ch * B, B)]],
                         sem, add=True)

    def chunk(i, carry):
        pltpu.async_copy(row_v, acc_s.at[dst_t.at[pl.ds((i + K) * B, B)]],
                         sem, add=True)
        s_desc(i).wait()
        return carry

    lax.fori_loop(0, NCH_ES - K, chunk, 0)
    for ch in range(NCH_ES - K, NCH_ES):
        s_desc(ch).wait()
    plsc.subcore_barrier()
    pltpu.sync_copy(acc_s.at[pl.ds(s * RPT, RPT)],
                    deg_out.at[pl.ds(c * NP + s * RPT, RPT)])


def _deg_call(dst_pad, zeros8, ones8):
    k = pl.kernel(
        _deg_body,
        out_type=jax.ShapeDtypeStruct((NC * NP, 8), _f32),
        mesh=_mesh(),
        scratch_types=[
            pltpu.VMEM((EPW_ES,), _i32),
            pltpu.VMEM((B, 8), _f32),
            pltpu.VMEM_SHARED((NP, 8), _f32),
            pltpu.SemaphoreType.DMA,
        ],
        compiler_params=_SC_PARAMS,
    )
    return k(dst_pad, zeros8, ones8)


def _gcn_body(src_hbm, dst_hbm, gtab, zeros16, acc_out,
              src_t, dst_t, rows0, rows1, acc_s, sg0, sg1, ss0, ss1):
    c = lax.axis_index("c")
    s = lax.axis_index("s")
    w = c * NS + s
    rows = (rows0, rows1)
    sg = (sg0, sg1)
    ss = (ss0, ss1)
    for k in range(RPT // B):
        pltpu.sync_copy(zeros16, acc_s.at[pl.ds(s * RPT + k * B, B)])
    pltpu.sync_copy(src_hbm.at[pl.ds(w * EPW_ES, EPW_ES)], src_t)
    pltpu.sync_copy(dst_hbm.at[pl.ds(w * EPW_ES, EPW_ES)], dst_t)
    plsc.subcore_barrier()

    def g_desc(ch, b):
        return pltpu.make_async_copy(gtab.at[src_t.at[pl.ds(ch * B, B)]],
                                     rows[b], sg[b])

    def s_desc(ch, b):
        return pltpu.make_async_copy(
            rows[b], acc_s.at[dst_t.at[pl.ds(ch * B, B)]], ss[b])

    def step(ch, b, issue_next, first=False):
        if not first:
            s_desc(ch - 1, 1 - b).wait()
        if issue_next:
            g_desc(ch + 1, 1 - b).start()
        g_desc(ch, b).wait()
        pltpu.async_copy(rows[b], acc_s.at[dst_t.at[pl.ds(ch * B, B)]],
                         ss[b], add=True)

    g_desc(0, 0).start()

    def pair(g, carry):
        @pl.when(g == 0)
        def _():
            g_desc(1, 1).start()
            g_desc(0, 0).wait()
            pltpu.async_copy(rows[0], acc_s.at[dst_t.at[pl.ds(0, B)]],
                             ss[0], add=True)

        @pl.when(g > 0)
        def _():
            step(2 * g, 0, True)

        step(2 * g + 1, 1, True)
        return carry

    lax.fori_loop(0, NCH_ES // 2, pair, 0)
    step(NCH_ES - 1, 0, False)
    s_desc(NCH_ES - 1, 0).wait()
    plsc.subcore_barrier()
    pltpu.sync_copy(acc_s.at[pl.ds(s * RPT, RPT)],
                    acc_out.at[pl.ds(c * NP + s * RPT, RPT)])


def _gcn_call(src_pad, dst_pad, gtab, zeros16):
    k = pl.kernel(
        _gcn_body,
        out_type=jax.ShapeDtypeStruct((NC * NP, 16), _f32),
        mesh=_mesh(),
        scratch_types=[
            pltpu.VMEM((EPW_ES,), _i32),
            pltpu.VMEM((EPW_ES,), _i32),
            pltpu.VMEM((B, 16), _f32),
            pltpu.VMEM((B, 16), _f32),
            pltpu.VMEM_SHARED((NP, 16), _f32),
            pltpu.SemaphoreType.DMA,
            pltpu.SemaphoreType.DMA,
            pltpu.SemaphoreType.DMA,
            pltpu.SemaphoreType.DMA,
        ],
        compiler_params=_SC_PARAMS,
    )
    return k(src_pad, dst_pad, gtab, zeros16)


def _gat_body(gsrc_hbm, dst_hbm, htab, altabd, shift_hbm, zeros72, acc_out,
              gidx_t, dst_t, rows0, rows1, drow0, drow1, send0, send1,
              shift_v, acc_s, sg0, sg1, ss0, ss1):
    c = lax.axis_index("c")
    s = lax.axis_index("s")
    rows = (rows0, rows1)
    drow = (drow0, drow1)
    send = (send0, send1)
    sg = (sg0, sg1)
    ss = (ss0, ss1)
    for k in range(RPT // B):
        pltpu.sync_copy(zeros72, acc_s.at[pl.ds(s * RPT + k * B, B)])
    pltpu.sync_copy(gsrc_hbm.at[pl.ds(c * ET + s * EPW_FS, EPW_FS)], gidx_t)
    pltpu.sync_copy(dst_hbm.at[pl.ds(s * EPW_FS, EPW_FS)], dst_t)
    pltpu.sync_copy(shift_hbm, shift_v)
    plsc.subcore_barrier()
    shv = shift_v[...]

    def g_pair(ch, b):
        return (pltpu.make_async_copy(htab.at[gidx_t.at[pl.ds(ch * B, B)]],
                                      rows[b], sg[b]),
                pltpu.make_async_copy(altabd.at[dst_t.at[pl.ds(ch * B, B)]],
                                      drow[b], sg[b]))

    def g_issue(ch, b):
        for d in g_pair(ch, b):
            d.start()

    def g_wait(ch, b):
        for d in g_pair(ch, b):
            d.wait()

    def s_desc(ch, b):
        return pltpu.make_async_copy(
            send[b], acc_s.at[dst_t.at[pl.ds(ch * B, B)]], ss[b])

    def compute(b):
        for jj in range(B):
            sr = rows[b][jj, pl.ds(56, 16)]
            dr = drow[b][jj, :]
            e = sr + dr
            e = jnp.maximum(e, 0.2 * e)
            ex = jnp.exp(e - shv)
            send[b][jj, pl.ds(56, 16)] = ex
            for h in range(4):
                exb = _lane_take(ex, _bc16(8 + 4 * c + h))
                send[b][jj, pl.ds(16 * h, 16)] = (
                    rows[b][jj, pl.ds(16 * h, 16)] * exb)

    g_issue(0, 0)

    def pair(g, carry):
        g_issue(2 * g + 1, 1)
        g_wait(2 * g, 0)

        @pl.when(g >= 1)
        def _():
            s_desc(2 * g - 2, 0).wait()

        compute(0)
        pltpu.async_copy(send[0], acc_s.at[dst_t.at[pl.ds((2 * g) * B, B)]],
                         ss[0], add=True)
        @pl.when(g < NCH_FS // 2 - 1)
        def _():
            g_issue(2 * g + 2, 0)

        g_wait(2 * g + 1, 1)

        @pl.when(g >= 1)
        def _():
            s_desc(2 * g - 1, 1).wait()

        compute(1)
        pltpu.async_copy(send[1],
                         acc_s.at[dst_t.at[pl.ds((2 * g + 1) * B, B)]],
                         ss[1], add=True)
        return carry

    lax.fori_loop(0, NCH_FS // 2, pair, 0)
    s_desc(NCH_FS - 2, 0).wait()
    s_desc(NCH_FS - 1, 1).wait()
    plsc.subcore_barrier()
    pltpu.sync_copy(acc_s.at[pl.ds(s * RPT, RPT)],
                    acc_out.at[pl.ds(c * NP + s * RPT, RPT)])


def _gat_call(gsrc_pad, dst_pad, htab, altabd, shift, zeros72):
    k = pl.kernel(
        _gat_body,
        out_type=jax.ShapeDtypeStruct((NC * NP, 72), _f32),
        mesh=_mesh(),
        scratch_types=[
            pltpu.VMEM((EPW_FS,), _i32),
            pltpu.VMEM((EPW_FS,), _i32),
            pltpu.VMEM((B, 72), _f32),
            pltpu.VMEM((B, 72), _f32),
            pltpu.VMEM((B, 16), _f32),
            pltpu.VMEM((B, 16), _f32),
            pltpu.VMEM((B, 72), _f32),
            pltpu.VMEM((B, 72), _f32),
            pltpu.VMEM((16,), _f32),
            pltpu.VMEM_SHARED((NP, 72), _f32),
            pltpu.SemaphoreType.DMA,
            pltpu.SemaphoreType.DMA,
            pltpu.SemaphoreType.DMA,
            pltpu.SemaphoreType.DMA,
        ],
        compiler_params=_SC_PARAMS,
    )
    return k(gsrc_pad, dst_pad, htab, altabd, shift, zeros72)


def _gat4_body(src_hbm, dst_hbm, htab4, altabd4, shift_hbm, zeros16, acc_out,
               src_t, dst_t, rows0, rows1, drow0, drow1, send0, send1,
               shift_v, acc_s, sg0, sg1, ss0, ss1):
    c = lax.axis_index("c")
    s = lax.axis_index("s")
    w = c * NS + s
    rows = (rows0, rows1)
    drow = (drow0, drow1)
    send = (send0, send1)
    sg = (sg0, sg1)
    ss = (ss0, ss1)
    for k in range(RPT // B):
        pltpu.sync_copy(zeros16, acc_s.at[pl.ds(s * RPT + k * B, B)])
    pltpu.sync_copy(src_hbm.at[pl.ds(w * EPW_ES, EPW_ES)], src_t)
    pltpu.sync_copy(dst_hbm.at[pl.ds(w * EPW_ES, EPW_ES)], dst_t)
    pltpu.sync_copy(shift_hbm, shift_v)
    plsc.subcore_barrier()
    shv = shift_v[...]
    lane3 = _bc16(3)

    def g_pair(ch, b):
        return (pltpu.make_async_copy(htab4.at[src_t.at[pl.ds(ch * B, B)]],
                                      rows[b], sg[b]),
                pltpu.make_async_copy(altabd4.at[dst_t.at[pl.ds(ch * B, B)]],
                                      drow[b], sg[b]))

    def g_issue(ch, b):
        for d in g_pair(ch, b):
            d.start()

    def g_wait(ch, b):
        for d in g_pair(ch, b):
            d.wait()

    def s_desc(ch, b):
        return pltpu.make_async_copy(
            send[b], acc_s.at[dst_t.at[pl.ds(ch * B, B)]], ss[b])

    def compute(b):
        for jj in range(B):
            sr = _lane_take(rows[b][jj, :], lane3)
            dr = drow[b][jj, :]
            e = sr + dr
            e = jnp.maximum(e, 0.2 * e)
            ex = jnp.exp(e - shv)
            send[b][jj, :] = rows[b][jj, :] * ex

    def step(ch, b, issue_next):
        if issue_next:
            g_issue(ch + 1, 1 - b)
        g_wait(ch, b)
        if isinstance(ch, int):
            if ch >= 2:
                s_desc(ch - 2, b).wait()
        else:
            @pl.when(ch >= 2)
            def _():
                s_desc(ch - 2, b).wait()

        compute(b)
        pltpu.async_copy(send[b], acc_s.at[dst_t.at[pl.ds(ch * B, B)]],
                         ss[b], add=True)

    g_issue(0, 0)
    NPAIR = NCH_ES // 2

    def pair(g, carry):
        step(2 * g, 0, True)
        step(2 * g + 1, 1, True)
        return carry

    lax.fori_loop(0, NPAIR, pair, 0)
    step(NCH_ES - 1, 0, False)
    s_desc(NCH_ES - 2, 1).wait()
    s_desc(NCH_ES - 1, 0).wait()
    plsc.subcore_barrier()
    pltpu.sync_copy(acc_s.at[pl.ds(s * RPT, RPT)],
                    acc_out.at[pl.ds(c * NP + s * RPT, RPT)])


def _gat4_call(src_pad, dst_pad, htab4, altabd4, shift4, zeros16):
    k = pl.kernel(
        _gat4_body,
        out_type=jax.ShapeDtypeStruct((NC * NP, 16), _f32),
        mesh=_mesh(),
        scratch_types=[
            pltpu.VMEM((EPW_ES,), _i32),
            pltpu.VMEM((EPW_ES,), _i32),
            pltpu.VMEM((B, 16), _f32),
            pltpu.VMEM((B, 16), _f32),
            pltpu.VMEM((B, 16), _f32),
            pltpu.VMEM((B, 16), _f32),
            pltpu.VMEM((B, 16), _f32),
            pltpu.VMEM((B, 16), _f32),
            pltpu.VMEM((16,), _f32),
            pltpu.VMEM_SHARED((NP, 16), _f32),
            pltpu.SemaphoreType.DMA,
            pltpu.SemaphoreType.DMA,
            pltpu.SemaphoreType.DMA,
            pltpu.SemaphoreType.DMA,
        ],
        compiler_params=_SC_PARAMS,
    )
    return k(src_pad, dst_pad, htab4, altabd4, shift4, zeros16)


def _elu(x):
    return jnp.where(x > 0, x, jnp.exp(jnp.minimum(x, 0.0)) - 1.0)


def _leaky(x):
    return jnp.maximum(x, 0.2 * x)


R0 = 2000
R = 1280
R5 = 1000


def _tc0a_body(x_ref, wg_ref, h0_ref):
    h0_ref[...] = x_ref[...] @ wg_ref[...]


def _tc0a_call(x, W_gcn):
    return pl.pallas_call(
        _tc0a_body,
        grid=(N // R0,),
        in_specs=[
            pl.BlockSpec((R0, D), lambda i: (i, 0)),
            pl.BlockSpec((D, 16), lambda i: (0, 0)),
        ],
        out_specs=pl.BlockSpec((R0, 16), lambda i: (i, 0)),
        out_shape=jax.ShapeDtypeStruct((NP, 16), _f32),
    )(x, W_gcn)


def _tc0b_body(h0_ref, deg_ref, g_ref):
    degs = deg_ref[0, :, 0:1] + deg_ref[1, :, 0:1]
    dinv = lax.rsqrt(jnp.maximum(degs, 1.0))
    g_ref[...] = h0_ref[...] * dinv


def _tc0b_call(h0, deg2):
    return pl.pallas_call(
        _tc0b_body,
        grid=(N // R0,),
        in_specs=[
            pl.BlockSpec((R0, 16), lambda i: (i, 0)),
            pl.BlockSpec((2, R0, 8), lambda i: (0, i, 0)),
        ],
        out_specs=pl.BlockSpec((R0, 16), lambda i: (i, 0)),
        out_shape=jax.ShapeDtypeStruct((NP, 16), _f32),
    )(h0, deg2)


def _attn_tail(i, h, As_ref, Ad_ref, htab_ref, altab_ref, shift_ref, mxs, mxd):
    als = h @ As_ref[...]
    ald = h @ Ad_ref[...]
    htab_ref[...] = jnp.stack(
        [jnp.concatenate([h[:, :64], als], axis=1),
         jnp.concatenate([h[:, 64:], als], axis=1)], axis=0)
    altab_ref[...] = jnp.concatenate([ald, ald], axis=1)

    @pl.when(i == 0)
    def _():
        mxs[...] = jnp.full((1, 8), -1e30, _f32)
        mxd[...] = jnp.full((1, 8), -1e30, _f32)

    rblk = als.shape[0]
    valid = (lax.broadcasted_iota(_i32, (rblk, 8), 0) + i * rblk) < N
    mxs[...] = jnp.maximum(mxs[...],
                           jnp.max(jnp.where(valid, als, -1e30), axis=0,
                                   keepdims=True))
    mxd[...] = jnp.maximum(mxd[...],
                           jnp.max(jnp.where(valid, ald, -1e30), axis=0,
                                   keepdims=True))
    sh = _leaky(mxs[...] + mxd[...])
    shift_ref[...] = jnp.concatenate([jnp.zeros((1, 8), _f32), sh], axis=1)


def _tc1_body(accg_ref, deg_ref, bg_ref, W_ref, As_ref, Ad_ref,
              htab_ref, altab_ref, shift_ref, mxs, mxd):
    i = pl.program_id(0)
    g = accg_ref[0] + accg_ref[1]
    degs = deg_ref[0, :, 0:1] + deg_ref[1, :, 0:1]
    dinv = lax.rsqrt(jnp.maximum(degs, 1.0))
    x1 = _elu(g * dinv + bg_ref[...])
    h = x1 @ W_ref[...]
    _attn_tail(i, h, As_ref, Ad_ref, htab_ref, altab_ref, shift_ref, mxs, mxd)


def _gat_outs():
    return dict(
        out_specs=[
            pl.BlockSpec((2, R, 72), lambda i: (0, i, 0)),
            pl.BlockSpec((R, 16), lambda i: (i, 0)),
            pl.BlockSpec((1, 16), lambda i: (0, 0)),
        ],
        out_shape=[
            jax.ShapeDtypeStruct((2, NP, 72), _f32),
            jax.ShapeDtypeStruct((NP, 16), _f32),
            jax.ShapeDtypeStruct((1, 16), _f32),
        ],
        scratch_shapes=[pltpu.VMEM((1, 8), _f32), pltpu.VMEM((1, 8), _f32)],
    )


def _tc1_call(accg2, deg2, b_gcn, W1, As1, Ad1):
    return pl.pallas_call(
        _tc1_body,
        grid=(NP // R,),
        in_specs=[
            pl.BlockSpec((2, R, 16), lambda i: (0, i, 0)),
            pl.BlockSpec((2, R, 8), lambda i: (0, i, 0)),
            pl.BlockSpec((1, 16), lambda i: (0, 0)),
            pl.BlockSpec((16, D), lambda i: (0, 0)),
            pl.BlockSpec((D, 8), lambda i: (0, 0)),
            pl.BlockSpec((D, 8), lambda i: (0, 0)),
        ],
        **_gat_outs(),
    )(accg2, deg2, b_gcn, W1, As1, Ad1)


def _xin_from_acc(acc_ref, b_ref, Rep4):
    xs = []
    for cc in range(2):
        f = acc_ref[cc, :, 0:64]
        den = acc_ref[cc, :, 64 + 4 * cc:68 + 4 * cc] @ Rep4
        xs.append(f / (den + 1e-16))
    return _elu(jnp.concatenate(xs, axis=1) + b_ref[...])


def _tcmid_body(rep_ref, acc_ref, b_ref, W_ref, As_ref, Ad_ref,
                htab_ref, altab_ref, shift_ref, mxs, mxd):
    i = pl.program_id(0)
    x = _xin_from_acc(acc_ref, b_ref, rep_ref[...])
    h = x @ W_ref[...]
    _attn_tail(i, h, As_ref, Ad_ref, htab_ref, altab_ref, shift_ref, mxs, mxd)


def _tcmid_call(rep4, acc2, b_prev, W, As, Ad):
    return pl.pallas_call(
        _tcmid_body,
        grid=(NP // R,),
        in_specs=[
            pl.BlockSpec((4, 64), lambda i: (0, 0)),
            pl.BlockSpec((2, R, 72), lambda i: (0, i, 0)),
            pl.BlockSpec((1, D), lambda i: (0, 0)),
            pl.BlockSpec((D, D), lambda i: (0, 0)),
            pl.BlockSpec((D, 8), lambda i: (0, 0)),
            pl.BlockSpec((D, 8), lambda i: (0, 0)),
        ],
        **_gat_outs(),
    )(rep4, acc2, b_prev, W, As, Ad)


def _tc4_body(rep_ref, acc_ref, b_ref, W_ref, As_ref, Ad_ref,
              htab_ref, altab_ref, shift_ref, mxs, mxd):
    i = pl.program_id(0)
    x = _xin_from_acc(acc_ref, b_ref, rep_ref[...])
    h4 = x @ W_ref[...]
    als = h4 @ As_ref[...]
    ald = h4 @ Ad_ref[...]
    rblk = h4.shape[0]
    htab_ref[...] = jnp.concatenate(
        [h4, jnp.ones((rblk, 1), _f32), als, jnp.zeros((rblk, 12), _f32)],
        axis=1)
    altab_ref[...] = jnp.concatenate([ald] * 16, axis=1)

    @pl.when(i == 0)
    def _():
        mxs[...] = jnp.full((1, 8), -1e30, _f32)
        mxd[...] = jnp.full((1, 8), -1e30, _f32)

    valid = (lax.broadcasted_iota(_i32, (rblk, 1), 0) + i * rblk) < N
    mxs[...] = jnp.maximum(
        mxs[...],
        jnp.max(jnp.where(valid, als, -1e30), axis=0, keepdims=True))
    mxd[...] = jnp.maximum(
        mxd[...],
        jnp.max(jnp.where(valid, ald, -1e30), axis=0, keepdims=True))
    sh = _leaky(mxs[...] + mxd[...])
    shift_ref[...] = jnp.concatenate(
        [sh[:, 0:1], jnp.zeros((1, 15), _f32)], axis=1)


def _tc4_call(rep4, acc2, b3, W4, As4, Ad4):
    return pl.pallas_call(
        _tc4_body,
        grid=(NP // R,),
        in_specs=[
            pl.BlockSpec((4, 64), lambda i: (0, 0)),
            pl.BlockSpec((2, R, 72), lambda i: (0, i, 0)),
            pl.BlockSpec((1, D), lambda i: (0, 0)),
            pl.BlockSpec((D, 2), lambda i: (0, 0)),
            pl.BlockSpec((2, 1), lambda i: (0, 0)),
            pl.BlockSpec((2, 1), lambda i: (0, 0)),
        ],
        out_specs=[
            pl.BlockSpec((R, 16), lambda i: (i, 0)),
            pl.BlockSpec((R, 16), lambda i: (i, 0)),
            pl.BlockSpec((1, 16), lambda i: (0, 0)),
        ],
        out_shape=[
            jax.ShapeDtypeStruct((NP, 16), _f32),
            jax.ShapeDtypeStruct((NP, 16), _f32),
            jax.ShapeDtypeStruct((1, 16), _f32),
        ],
        scratch_shapes=[pltpu.VMEM((1, 8), _f32), pltpu.VMEM((1, 8), _f32)],
    )(rep4, acc2, b3, W4, As4, Ad4)


def _tc5_body(accU_ref, accI_ref, b_ref, out_ref):
    def node_h(a):
        f = a[0, :, 0:2] + a[1, :, 0:2]
        den = a[0, :, 2:3] + a[1, :, 2:3]
        return _elu(f / (den + 1e-16) + b_ref[...])

    z = jnp.concatenate([node_h(accU_ref[...]), node_h(accI_ref[...])], axis=1)
    m = jnp.max(z, axis=1, keepdims=True)
    lse = jnp.log(jnp.sum(jnp.exp(z - m), axis=1, keepdims=True)) + m
    out_ref[...] = z - lse


def _tc5_call(acc42, b4):
    return pl.pallas_call(
        _tc5_body,
        grid=(N // 2 // R5,),
        in_specs=[
            pl.BlockSpec((2, R5, 16), lambda i: (0, i, 0)),
            pl.BlockSpec((2, R5, 16), lambda i: (0, i + 5, 0)),
            pl.BlockSpec((1, 2), lambda i: (0, 0)),
        ],
        out_specs=pl.BlockSpec((R5, 4), lambda i: (i, 0)),
        out_shape=jax.ShapeDtypeStruct((N // 2, 4), _f32),
    )(acc42, acc42, b4)


def _head_proj(a):
    H, C = a.shape
    m = jnp.zeros((H * C, H), _f32)
    hh = jnp.arange(H * C) // C
    return m.at[jnp.arange(H * C), hh].set(a.reshape(-1))


def kernel(x, edge_index, batch, W_gcn, b_gcn, W1, as1, ad1, b1,
           W2, as2, ad2, b2, W3, as3, ad3, b3, W4, as4, ad4, b4):
    ar = jnp.arange(N, dtype=_i32)
    npad = ET - E - N
    src_pad = jnp.concatenate(
        [edge_index[0], ar, jnp.full((npad,), N, _i32)])
    dst_pad = jnp.concatenate(
        [edge_index[1], ar, jnp.full((npad,), N, _i32)])

    zeros8 = jnp.zeros((B, 8), _f32)
    ones8 = zeros8.at[:, 0].set(1.0)
    zeros16 = jnp.zeros((B, 16), _f32)
    zeros72 = jnp.zeros((B, 72), _f32)
    rep4 = jnp.repeat(jnp.eye(4, dtype=_f32), 16, axis=1)

    As1, Ad1 = _head_proj(as1), _head_proj(ad1)
    As2, Ad2 = _head_proj(as2), _head_proj(ad2)
    As3, Ad3 = _head_proj(as3), _head_proj(ad3)
    As4, Ad4 = _head_proj(as4), _head_proj(ad4)

    h0 = _tc0a_call(x, W_gcn)
    deg2 = _deg_call(dst_pad, zeros8, ones8).reshape(2, NP, 8)
    gtab = _tc0b_call(h0, deg2)
    accg = _gcn_call(src_pad, dst_pad, gtab, zeros16).reshape(2, NP, 16)

    gsrc_pad = jnp.concatenate([src_pad, src_pad + NP])

    htab, altab, shift = _tc1_call(accg, deg2, b_gcn.reshape(1, 16),
                                   W1, As1, Ad1)
    acc1 = _gat_call(gsrc_pad, dst_pad, htab.reshape(2 * NP, 72),
                     altab, shift.reshape(16), zeros72)

    htab, altab, shift = _tcmid_call(rep4, acc1.reshape(2, NP, 72),
                                     b1.reshape(1, D), W2, As2, Ad2)
    acc2 = _gat_call(gsrc_pad, dst_pad, htab.reshape(2 * NP, 72),
                     altab, shift.reshape(16), zeros72)
    htab, altab, shift = _tcmid_call(rep4, acc2.reshape(2, NP, 72),
                                     b2.reshape(1, D), W3, As3, Ad3)
    acc3 = _gat_call(gsrc_pad, dst_pad, htab.reshape(2 * NP, 72),
                     altab, shift.reshape(16), zeros72)

    htab4, altab4, shift4 = _tc4_call(rep4, acc3.reshape(2, NP, 72),
                                      b3.reshape(1, D), W4, As4, Ad4)
    acc4 = _gat4_call(src_pad, dst_pad, htab4, altab4,
                      shift4.reshape(16), zeros16)

    return _tc5_call(acc4.reshape(2, NP, 16), b4.reshape(1, 2))


# speedup vs baseline: 1.0905x; 1.0019x over previous
import jax
import jax.numpy as jnp
from jax import lax
from jax.experimental import pallas as pl
from jax.experimental.pallas import tpu as pltpu
from jax.experimental.pallas import tpu_sc as plsc

N = 10000
D = 128
E = 320000
NC, NS, L = 2, 16, 16
NW = NC * NS
NP = 10240
RPT = NP // NS
B = 128
ET = 331776
EPW_ES = ET // NW
EPW_FS = ET // NS
NCH_ES = EPW_ES // B
NCH_FS = EPW_FS // B

_f32 = jnp.float32
_i32 = jnp.int32

_SC_PARAMS = pltpu.CompilerParams(use_tc_tiling_on_sc=False)


def _mesh():
    return plsc.VectorSubcoreMesh(core_axis_name="c", subcore_axis_name="s",
                                  num_cores=NC, num_subcores=NS)


def _lane_take(x, idx):
    dn = lax.GatherDimensionNumbers(offset_dims=(), collapsed_slice_dims=(0,),
                                    start_index_map=(0,))
    return lax.gather(x, idx[:, None], dn, (1,),
                      mode=lax.GatherScatterMode.PROMISE_IN_BOUNDS)


def _bc16(v):
    return jnp.zeros((16,), _i32) + v


def _deg_body(dst_hbm, zeros8, ones8, deg_out, dst_t, row_v, acc_s, sem):
    c = lax.axis_index("c")
    s = lax.axis_index("s")
    w = c * NS + s
    for k in range(RPT // B):
        pltpu.sync_copy(zeros8, acc_s.at[pl.ds(s * RPT + k * B, B)])
    pltpu.sync_copy(dst_hbm.at[pl.ds(w * EPW_ES, EPW_ES)], dst_t)
    pltpu.sync_copy(ones8, row_v)
    plsc.subcore_barrier()

    def s_desc(ch):
        return pltpu.make_async_copy(
            row_v, acc_s.at[dst_t.at[pl.ds(ch * B, B)]], sem)

    K = 3
    for ch in range(K):
        pltpu.async_copy(row_v, acc_s.at[dst_t.at[pl.ds(ch * B, B)]],
                         sem, add=True)

    def chunk(i, carry):
        pltpu.async_copy(row_v, acc_s.at[dst_t.at[pl.ds((i + K) * B, B)]],
                         sem, add=True)
        s_desc(i).wait()
        return carry

    lax.fori_loop(0, NCH_ES - K, chunk, 0)
    for ch in range(NCH_ES - K, NCH_ES):
        s_desc(ch).wait()
    plsc.subcore_barrier()
    pltpu.sync_copy(acc_s.at[pl.ds(s * RPT, RPT)],
                    deg_out.at[pl.ds(c * NP + s * RPT, RPT)])


def _deg_call(dst_pad, zeros8, ones8):
    k = pl.kernel(
        _deg_body,
        out_type=jax.ShapeDtypeStruct((NC * NP, 8), _f32),
        mesh=_mesh(),
        scratch_types=[
            pltpu.VMEM((EPW_ES,), _i32),
            pltpu.VMEM((B, 8), _f32),
            pltpu.VMEM_SHARED((NP, 8), _f32),
            pltpu.SemaphoreType.DMA,
        ],
        compiler_params=_SC_PARAMS,
    )
    return k(dst_pad, zeros8, ones8)


def _gcn_body(src_hbm, dst_hbm, gtab, zeros16, acc_out,
              src_t, dst_t, rows0, rows1, acc_s, sg0, sg1, ss0, ss1):
    c = lax.axis_index("c")
    s = lax.axis_index("s")
    w = c * NS + s
    rows = (rows0, rows1)
    sg = (sg0, sg1)
    ss = (ss0, ss1)
    for k in range(RPT // B):
        pltpu.sync_copy(zeros16, acc_s.at[pl.ds(s * RPT + k * B, B)])
    pltpu.sync_copy(src_hbm.at[pl.ds(w * EPW_ES, EPW_ES)], src_t)
    pltpu.sync_copy(dst_hbm.at[pl.ds(w * EPW_ES, EPW_ES)], dst_t)
    plsc.subcore_barrier()

    def g_desc(ch, b):
        return pltpu.make_async_copy(gtab.at[src_t.at[pl.ds(ch * B, B)]],
                                     rows[b], sg[b])

    def s_desc(ch, b):
        return pltpu.make_async_copy(
            rows[b], acc_s.at[dst_t.at[pl.ds(ch * B, B)]], ss[b])

    def step(ch, b, issue_next, first=False):
        if not first:
            s_desc(ch - 1, 1 - b).wait()
        if issue_next:
            g_desc(ch + 1, 1 - b).start()
        g_desc(ch, b).wait()
        pltpu.async_copy(rows[b], acc_s.at[dst_t.at[pl.ds(ch * B, B)]],
                         ss[b], add=True)

    g_desc(0, 0).start()

    def pair(g, carry):
        @pl.when(g == 0)
        def _():
            g_desc(1, 1).start()
            g_desc(0, 0).wait()
            pltpu.async_copy(rows[0], acc_s.at[dst_t.at[pl.ds(0, B)]],
                             ss[0], add=True)

        @pl.when(g > 0)
        def _():
            step(2 * g, 0, True)

        step(2 * g + 1, 1, True)
        return carry

    lax.fori_loop(0, NCH_ES // 2, pair, 0)
    step(NCH_ES - 1, 0, False)
    s_desc(NCH_ES - 1, 0).wait()
    plsc.subcore_barrier()
    pltpu.sync_copy(acc_s.at[pl.ds(s * RPT, RPT)],
                    acc_out.at[pl.ds(c * NP + s * RPT, RPT)])


def _gcn_call(src_pad, dst_pad, gtab, zeros16):
    k = pl.kernel(
        _gcn_body,
        out_type=jax.ShapeDtypeStruct((NC * NP, 16), _f32),
        mesh=_mesh(),
        scratch_types=[
            pltpu.VMEM((EPW_ES,), _i32),
            pltpu.VMEM((EPW_ES,), _i32),
            pltpu.VMEM((B, 16), _f32),
            pltpu.VMEM((B, 16), _f32),
            pltpu.VMEM_SHARED((NP, 16), _f32),
            pltpu.SemaphoreType.DMA,
            pltpu.SemaphoreType.DMA,
            pltpu.SemaphoreType.DMA,
            pltpu.SemaphoreType.DMA,
        ],
        compiler_params=_SC_PARAMS,
    )
    return k(src_pad, dst_pad, gtab, zeros16)


def _gat_body(gsrc_hbm, dst_hbm, htab, altabd, shift_hbm, zeros72, acc_out,
              gidx_t, dst_t, rows0, rows1, drow0, drow1, send0, send1,
              shift_v, acc_s, sg0, sg1, ss0, ss1):
    c = lax.axis_index("c")
    s = lax.axis_index("s")
    rows = (rows0, rows1)
    drow = (drow0, drow1)
    send = (send0, send1)
    sg = (sg0, sg1)
    ss = (ss0, ss1)
    for k in range(RPT // B):
        pltpu.sync_copy(zeros72, acc_s.at[pl.ds(s * RPT + k * B, B)])
    pltpu.sync_copy(gsrc_hbm.at[pl.ds(c * ET + s * EPW_FS, EPW_FS)], gidx_t)
    pltpu.sync_copy(dst_hbm.at[pl.ds(s * EPW_FS, EPW_FS)], dst_t)
    pltpu.sync_copy(shift_hbm, shift_v)
    plsc.subcore_barrier()
    shv = shift_v[...]

    def g_pair(ch, b):
        return (pltpu.make_async_copy(htab.at[gidx_t.at[pl.ds(ch * B, B)]],
                                      rows[b], sg[b]),
                pltpu.make_async_copy(altabd.at[dst_t.at[pl.ds(ch * B, B)]],
                                      drow[b], sg[b]))

    def g_issue(ch, b):
        for d in g_pair(ch, b):
            d.start()

    def g_wait(ch, b):
        for d in g_pair(ch, b):
            d.wait()

    def s_desc(ch, b):
        return pltpu.make_async_copy(
            send[b], acc_s.at[dst_t.at[pl.ds(ch * B, B)]], ss[b])

    def compute(b):
        for jj in range(B):
            sr = rows[b][jj, pl.ds(56, 16)]
            dr = drow[b][jj, :]
            e = sr + dr
            e = jnp.maximum(e, 0.2 * e)
            ex = jnp.exp(e - shv)
            send[b][jj, pl.ds(56, 16)] = ex
            for h in range(4):
                exb = _lane_take(ex, _bc16(8 + 4 * c + h))
                send[b][jj, pl.ds(16 * h, 16)] = (
                    rows[b][jj, pl.ds(16 * h, 16)] * exb)

    g_issue(0, 0)

    def pair(g, carry):
        g_issue(2 * g + 1, 1)
        g_wait(2 * g, 0)

        @pl.when(g >= 1)
        def _():
            s_desc(2 * g - 2, 0).wait()

        compute(0)
        pltpu.async_copy(send[0], acc_s.at[dst_t.at[pl.ds((2 * g) * B, B)]],
                         ss[0], add=True)
        @pl.when(g < NCH_FS // 2 - 1)
        def _():
            g_issue(2 * g + 2, 0)

        g_wait(2 * g + 1, 1)

        @pl.when(g >= 1)
        def _():
            s_desc(2 * g - 1, 1).wait()

        compute(1)
        pltpu.async_copy(send[1],
                         acc_s.at[dst_t.at[pl.ds((2 * g + 1) * B, B)]],
                         ss[1], add=True)
        return carry

    lax.fori_loop(0, NCH_FS // 2, pair, 0)
    s_desc(NCH_FS - 2, 0).wait()
    s_desc(NCH_FS - 1, 1).wait()
    plsc.subcore_barrier()
    pltpu.sync_copy(acc_s.at[pl.ds(s * RPT, RPT)],
                    acc_out.at[pl.ds(c * NP + s * RPT, RPT)])


def _gat_call(gsrc_pad, dst_pad, htab, altabd, shift, zeros72):
    k = pl.kernel(
        _gat_body,
        out_type=jax.ShapeDtypeStruct((NC * NP, 72), _f32),
        mesh=_mesh(),
        scratch_types=[
            pltpu.VMEM((EPW_FS,), _i32),
            pltpu.VMEM((EPW_FS,), _i32),
            pltpu.VMEM((B, 72), _f32),
            pltpu.VMEM((B, 72), _f32),
            pltpu.VMEM((B, 16), _f32),
            pltpu.VMEM((B, 16), _f32),
            pltpu.VMEM((B, 72), _f32),
            pltpu.VMEM((B, 72), _f32),
            pltpu.VMEM((16,), _f32),
            pltpu.VMEM_SHARED((NP, 72), _f32),
            pltpu.SemaphoreType.DMA,
            pltpu.SemaphoreType.DMA,
            pltpu.SemaphoreType.DMA,
            pltpu.SemaphoreType.DMA,
        ],
        compiler_params=_SC_PARAMS,
    )
    return k(gsrc_pad, dst_pad, htab, altabd, shift, zeros72)


def _gat4_body(src_hbm, dst_hbm, htab4, altabd4, shift_hbm, zeros16, acc_out,
               src_t, dst_t, rows0, rows1, drow0, drow1, send0, send1,
               shift_v, acc_s, sg0, sg1, ss0, ss1):
    c = lax.axis_index("c")
    s = lax.axis_index("s")
    w = c * NS + s
    rows = (rows0, rows1)
    drow = (drow0, drow1)
    send = (send0, send1)
    sg = (sg0, sg1)
    ss = (ss0, ss1)
    for k in range(RPT // B):
        pltpu.sync_copy(zeros16, acc_s.at[pl.ds(s * RPT + k * B, B)])
    pltpu.sync_copy(src_hbm.at[pl.ds(w * EPW_ES, EPW_ES)], src_t)
    pltpu.sync_copy(dst_hbm.at[pl.ds(w * EPW_ES, EPW_ES)], dst_t)
    pltpu.sync_copy(shift_hbm, shift_v)
    plsc.subcore_barrier()
    shv = shift_v[...]
    lane3 = _bc16(3)

    def g_pair(ch, b):
        return (pltpu.make_async_copy(htab4.at[src_t.at[pl.ds(ch * B, B)]],
                                      rows[b], sg[b]),
                pltpu.make_async_copy(altabd4.at[dst_t.at[pl.ds(ch * B, B)]],
                                      drow[b], sg[b]))

    def g_issue(ch, b):
        for d in g_pair(ch, b):
            d.start()

    def g_wait(ch, b):
        for d in g_pair(ch, b):
            d.wait()

    def s_desc(ch, b):
        return pltpu.make_async_copy(
            send[b], acc_s.at[dst_t.at[pl.ds(ch * B, B)]], ss[b])

    def compute(b):
        for jj in range(B):
            sr = _lane_take(rows[b][jj, :], lane3)
            dr = drow[b][jj, :]
            e = sr + dr
            e = jnp.maximum(e, 0.2 * e)
            ex = jnp.exp(e - shv)
            send[b][jj, :] = rows[b][jj, :] * ex

    def step(ch, b, issue_next):
        if issue_next:
            g_issue(ch + 1, 1 - b)
        g_wait(ch, b)
        if isinstance(ch, int):
            if ch >= 2:
                s_desc(ch - 2, b).wait()
        else:
            @pl.when(ch >= 2)
            def _():
                s_desc(ch - 2, b).wait()

        compute(b)
        pltpu.async_copy(send[b], acc_s.at[dst_t.at[pl.ds(ch * B, B)]],
                         ss[b], add=True)

    g_issue(0, 0)
    NPAIR = NCH_ES // 2

    def pair(g, carry):
        step(2 * g, 0, True)
        step(2 * g + 1, 1, True)
        return carry

    lax.fori_loop(0, NPAIR, pair, 0)
    step(NCH_ES - 1, 0, False)
    s_desc(NCH_ES - 2, 1).wait()
    s_desc(NCH_ES - 1, 0).wait()
    plsc.subcore_barrier()
    pltpu.sync_copy(acc_s.at[pl.ds(s * RPT, RPT)],
                    acc_out.at[pl.ds(c * NP + s * RPT, RPT)])


def _gat4_call(src_pad, dst_pad, htab4, altabd4, shift4, zeros16):
    k = pl.kernel(
        _gat4_body,
        out_type=jax.ShapeDtypeStruct((NC * NP, 16), _f32),
        mesh=_mesh(),
        scratch_types=[
            pltpu.VMEM((EPW_ES,), _i32),
            pltpu.VMEM((EPW_ES,), _i32),
            pltpu.VMEM((B, 16), _f32),
            pltpu.VMEM((B, 16), _f32),
            pltpu.VMEM((B, 16), _f32),
            pltpu.VMEM((B, 16), _f32),
            pltpu.VMEM((B, 16), _f32),
            pltpu.VMEM((B, 16), _f32),
            pltpu.VMEM((16,), _f32),
            pltpu.VMEM_SHARED((NP, 16), _f32),
            pltpu.SemaphoreType.DMA,
            pltpu.SemaphoreType.DMA,
            pltpu.SemaphoreType.DMA,
            pltpu.SemaphoreType.DMA,
        ],
        compiler_params=_SC_PARAMS,
    )
    return k(src_pad, dst_pad, htab4, altabd4, shift4, zeros16)


def _elu(x):
    return jnp.where(x > 0, x, jnp.exp(jnp.minimum(x, 0.0)) - 1.0)


def _leaky(x):
    return jnp.maximum(x, 0.2 * x)


R0 = 2000
R = 1280
R5 = 1000


def _tc0a_body(x_ref, wg_ref, h0_ref):
    h0_ref[...] = x_ref[...] @ wg_ref[...]


def _tc0a_call(x, W_gcn):
    return pl.pallas_call(
        _tc0a_body,
        grid=(N // R0,),
        in_specs=[
            pl.BlockSpec((R0, D), lambda i: (i, 0)),
            pl.BlockSpec((D, 16), lambda i: (0, 0)),
        ],
        out_specs=pl.BlockSpec((R0, 16), lambda i: (i, 0)),
        out_shape=jax.ShapeDtypeStruct((NP, 16), _f32),
    )(x, W_gcn)


def _tc0b_body(h0_ref, deg_ref, g_ref):
    degs = deg_ref[0, :, 0:1] + deg_ref[1, :, 0:1]
    dinv = lax.rsqrt(jnp.maximum(degs, 1.0))
    g_ref[...] = h0_ref[...] * dinv


def _tc0b_call(h0, deg2):
    return pl.pallas_call(
        _tc0b_body,
        grid=(N // R0,),
        in_specs=[
            pl.BlockSpec((R0, 16), lambda i: (i, 0)),
            pl.BlockSpec((2, R0, 8), lambda i: (0, i, 0)),
        ],
        out_specs=pl.BlockSpec((R0, 16), lambda i: (i, 0)),
        out_shape=jax.ShapeDtypeStruct((NP, 16), _f32),
    )(h0, deg2)


def _attn_tail(i, h, As_ref, Ad_ref, htab_ref, altab_ref, shift_ref, mxs, mxd):
    als = h @ As_ref[...]
    ald = h @ Ad_ref[...]
    htab_ref[...] = jnp.stack(
        [jnp.concatenate([h[:, :64], als], axis=1),
         jnp.concatenate([h[:, 64:], als], axis=1)], axis=0)
    altab_ref[...] = jnp.concatenate([ald, ald], axis=1)

    @pl.when(i == 0)
    def _():
        mxs[...] = jnp.full((1, 8), -1e30, _f32)
        mxd[...] = jnp.full((1, 8), -1e30, _f32)

    rblk = als.shape[0]
    valid = (lax.broadcasted_iota(_i32, (rblk, 8), 0) + i * rblk) < N
    mxs[...] = jnp.maximum(mxs[...],
                           jnp.max(jnp.where(valid, als, -1e30), axis=0,
                                   keepdims=True))
    mxd[...] = jnp.maximum(mxd[...],
                           jnp.max(jnp.where(valid, ald, -1e30), axis=0,
                                   keepdims=True))
    sh = _leaky(mxs[...] + mxd[...])
    shift_ref[...] = jnp.concatenate([jnp.zeros((1, 8), _f32), sh], axis=1)


def _tc1_body(accg_ref, deg_ref, bg_ref, W_ref, As_ref, Ad_ref,
              htab_ref, altab_ref, shift_ref, mxs, mxd):
    i = pl.program_id(0)
    g = accg_ref[0] + accg_ref[1]
    degs = deg_ref[0, :, 0:1] + deg_ref[1, :, 0:1]
    dinv = lax.rsqrt(jnp.maximum(degs, 1.0))
    x1 = _elu(g * dinv + bg_ref[...])
    h = x1 @ W_ref[...]
    _attn_tail(i, h, As_ref, Ad_ref, htab_ref, altab_ref, shift_ref, mxs, mxd)


def _gat_outs():
    return dict(
        out_specs=[
            pl.BlockSpec((2, R, 72), lambda i: (0, i, 0)),
            pl.BlockSpec((R, 16), lambda i: (i, 0)),
            pl.BlockSpec((1, 16), lambda i: (0, 0)),
        ],
        out_shape=[
            jax.ShapeDtypeStruct((2, NP, 72), _f32),
            jax.ShapeDtypeStruct((NP, 16), _f32),
            jax.ShapeDtypeStruct((1, 16), _f32),
        ],
        scratch_shapes=[pltpu.VMEM((1, 8), _f32), pltpu.VMEM((1, 8), _f32)],
    )


def _tc1_call(accg2, deg2, b_gcn, W1, As1, Ad1):
    return pl.pallas_call(
        _tc1_body,
        grid=(NP // R,),
        in_specs=[
            pl.BlockSpec((2, R, 16), lambda i: (0, i, 0)),
            pl.BlockSpec((2, R, 8), lambda i: (0, i, 0)),
            pl.BlockSpec((1, 16), lambda i: (0, 0)),
            pl.BlockSpec((16, D), lambda i: (0, 0)),
            pl.BlockSpec((D, 8), lambda i: (0, 0)),
            pl.BlockSpec((D, 8), lambda i: (0, 0)),
        ],
        **_gat_outs(),
    )(accg2, deg2, b_gcn, W1, As1, Ad1)


def _xin_from_acc(acc_ref, b_ref, Rep4):
    xs = []
    for cc in range(2):
        f = acc_ref[cc, :, 0:64]
        den = acc_ref[cc, :, 64 + 4 * cc:68 + 4 * cc] @ Rep4
        xs.append(f / (den + 1e-16))
    return _elu(jnp.concatenate(xs, axis=1) + b_ref[...])


def _tcmid_body(rep_ref, acc_ref, b_ref, W_ref, As_ref, Ad_ref,
                htab_ref, altab_ref, shift_ref, mxs, mxd):
    i = pl.program_id(0)
    x = _xin_from_acc(acc_ref, b_ref, rep_ref[...])
    h = x @ W_ref[...]
    _attn_tail(i, h, As_ref, Ad_ref, htab_ref, altab_ref, shift_ref, mxs, mxd)


def _tcmid_call(rep4, acc2, b_prev, W, As, Ad):
    return pl.pallas_call(
        _tcmid_body,
        grid=(NP // R,),
        in_specs=[
            pl.BlockSpec((4, 64), lambda i: (0, 0)),
            pl.BlockSpec((2, R, 72), lambda i: (0, i, 0)),
            pl.BlockSpec((1, D), lambda i: (0, 0)),
            pl.BlockSpec((D, D), lambda i: (0, 0)),
            pl.BlockSpec((D, 8), lambda i: (0, 0)),
            pl.BlockSpec((D, 8), lambda i: (0, 0)),
        ],
        **_gat_outs(),
    )(rep4, acc2, b_prev, W, As, Ad)


def _tc4_body(rep_ref, acc_ref, b_ref, W_ref, As_ref, Ad_ref,
              htab_ref, altab_ref, shift_ref, mxs, mxd):
    i = pl.program_id(0)
    x = _xin_from_acc(acc_ref, b_ref, rep_ref[...])
    h4 = x @ W_ref[...]
    als = h4 @ As_ref[...]
    ald = h4 @ Ad_ref[...]
    rblk = h4.shape[0]
    htab_ref[...] = jnp.concatenate(
        [h4, jnp.ones((rblk, 1), _f32), als, jnp.zeros((rblk, 12), _f32)],
        axis=1)
    altab_ref[...] = jnp.concatenate([ald] * 16, axis=1)

    @pl.when(i == 0)
    def _():
        mxs[...] = jnp.full((1, 8), -1e30, _f32)
        mxd[...] = jnp.full((1, 8), -1e30, _f32)

    valid = (lax.broadcasted_iota(_i32, (rblk, 1), 0) + i * rblk) < N
    mxs[...] = jnp.maximum(
        mxs[...],
        jnp.max(jnp.where(valid, als, -1e30), axis=0, keepdims=True))
    mxd[...] = jnp.maximum(
        mxd[...],
        jnp.max(jnp.where(valid, ald, -1e30), axis=0, keepdims=True))
    sh = _leaky(mxs[...] + mxd[...])
    shift_ref[...] = jnp.concatenate(
        [sh[:, 0:1], jnp.zeros((1, 15), _f32)], axis=1)


def _tc4_call(rep4, acc2, b3, W4, As4, Ad4):
    return pl.pallas_call(
        _tc4_body,
        grid=(NP // R,),
        in_specs=[
            pl.BlockSpec((4, 64), lambda i: (0, 0)),
            pl.BlockSpec((2, R, 72), lambda i: (0, i, 0)),
            pl.BlockSpec((1, D), lambda i: (0, 0)),
            pl.BlockSpec((D, 2), lambda i: (0, 0)),
            pl.BlockSpec((2, 1), lambda i: (0, 0)),
            pl.BlockSpec((2, 1), lambda i: (0, 0)),
        ],
        out_specs=[
            pl.BlockSpec((R, 16), lambda i: (i, 0)),
            pl.BlockSpec((R, 16), lambda i: (i, 0)),
            pl.BlockSpec((1, 16), lambda i: (0, 0)),
        ],
        out_shape=[
            jax.ShapeDtypeStruct((NP, 16), _f32),
            jax.ShapeDtypeStruct((NP, 16), _f32),
            jax.ShapeDtypeStruct((1, 16), _f32),
        ],
        scratch_shapes=[pltpu.VMEM((1, 8), _f32), pltpu.VMEM((1, 8), _f32)],
    )(rep4, acc2, b3, W4, As4, Ad4)


def _tc5_body(accU_ref, accI_ref, b_ref, out_ref):
    def node_h(a):
        f = a[0, :, 0:2] + a[1, :, 0:2]
        den = a[0, :, 2:3] + a[1, :, 2:3]
        return _elu(f / (den + 1e-16) + b_ref[...])

    z = jnp.concatenate([node_h(accU_ref[...]), node_h(accI_ref[...])], axis=1)
    m = jnp.max(z, axis=1, keepdims=True)
    lse = jnp.log(jnp.sum(jnp.exp(z - m), axis=1, keepdims=True)) + m
    out_ref[...] = z - lse


def _tc5_call(acc42, b4):
    return pl.pallas_call(
        _tc5_body,
        grid=(N // 2 // R5,),
        in_specs=[
            pl.BlockSpec((2, R5, 16), lambda i: (0, i, 0)),
            pl.BlockSpec((2, R5, 16), lambda i: (0, i + 5, 0)),
            pl.BlockSpec((1, 2), lambda i: (0, 0)),
        ],
        out_specs=pl.BlockSpec((R5, 4), lambda i: (i, 0)),
        out_shape=jax.ShapeDtypeStruct((N // 2, 4), _f32),
    )(acc42, acc42, b4)


def _head_proj(a):
    H, C = a.shape
    m = jnp.zeros((H * C, H), _f32)
    hh = jnp.arange(H * C) // C
    return m.at[jnp.arange(H * C), hh].set(a.reshape(-1))


def kernel(x, edge_index, batch, W_gcn, b_gcn, W1, as1, ad1, b1,
           W2, as2, ad2, b2, W3, as3, ad3, b3, W4, as4, ad4, b4):
    ar = jnp.arange(N, dtype=_i32)
    npad = ET - E - N
    src_pad = jnp.concatenate(
        [edge_index[0], ar, jnp.full((npad,), N, _i32)])
    dst_pad = jnp.concatenate(
        [edge_index[1], ar, jnp.full((npad,), N, _i32)])

    zeros8 = jnp.zeros((B, 8), _f32)
    ones8 = zeros8.at[:, 0].set(1.0)
    zeros16 = jnp.zeros((B, 16), _f32)
    zeros72 = jnp.zeros((B, 72), _f32)
    rep4 = jnp.repeat(jnp.eye(4, dtype=_f32), 16, axis=1)

    As1, Ad1 = _head_proj(as1), _head_proj(ad1)
    As2, Ad2 = _head_proj(as2), _head_proj(ad2)
    As3, Ad3 = _head_proj(as3), _head_proj(ad3)
    As4, Ad4 = _head_proj(as4), _head_proj(ad4)

    deg2 = _deg_call(dst_pad, zeros8, ones8).reshape(2, NP, 8)
    gtab = _tc0b_call(_tc0a_call(x, W_gcn), deg2)
    accg = _gcn_call(src_pad, dst_pad, gtab, zeros16).reshape(2, NP, 16)

    gsrc_pad = jnp.concatenate([src_pad, src_pad + NP])

    htab, altab, shift = _tc1_call(accg, deg2, b_gcn.reshape(1, 16),
                                   W1, As1, Ad1)
    acc1 = _gat_call(gsrc_pad, dst_pad, htab.reshape(2 * NP, 72),
                     altab, shift.reshape(16), zeros72)

    htab, altab, shift = _tcmid_call(rep4, acc1.reshape(2, NP, 72),
                                     b1.reshape(1, D), W2, As2, Ad2)
    acc2 = _gat_call(gsrc_pad, dst_pad, htab.reshape(2 * NP, 72),
                     altab, shift.reshape(16), zeros72)
    htab, altab, shift = _tcmid_call(rep4, acc2.reshape(2, NP, 72),
                                     b2.reshape(1, D), W3, As3, Ad3)
    acc3 = _gat_call(gsrc_pad, dst_pad, htab.reshape(2 * NP, 72),
                     altab, shift.reshape(16), zeros72)

    htab4, altab4, shift4 = _tc4_call(rep4, acc3.reshape(2, NP, 72),
                                      b3.reshape(1, D), W4, As4, Ad4)
    acc4 = _gat4_call(src_pad, dst_pad, htab4, altab4,
                      shift4.reshape(16), zeros16)

    return _tc5_call(acc4.reshape(2, NP, 16), b4.reshape(1, 2))
